# baseline (speedup 1.0000x reference)
import functools
import jax
import jax.numpy as jnp
from jax import lax
from jax.experimental import pallas as pl
from jax.experimental.pallas import tpu as pltpu

B = 4
N0 = 4096
N1 = 2048
N2 = 1024
C = 128
K = 16


def _d1_body(pos8_ref, nposT_ref, f_ref, w10_ref, d_ref, z_ref):
    p = pos8_ref[0]
    nT = nposT_ref[0]
    d = jnp.zeros(d_ref.shape[1:], jnp.float32)
    for c in range(3):
        diff = p[:, c:c + 1] - nT[c:c + 1, :]
        d = d + diff * diff
    d_ref[0] = d
    z_ref[0] = jnp.dot(f_ref[0], w10_ref[...], preferred_element_type=jnp.float32, precision=lax.Precision.HIGHEST)


def _call_d1(pos8, nposT, f_pad, w10p):
    return pl.pallas_call(
        _d1_body,
        grid=(B, N0 // 512),
        in_specs=[
            pl.BlockSpec((1, 512, 8), lambda b, j: (b, j, 0)),
            pl.BlockSpec((1, 8, N1), lambda b, j: (b, 0, 0)),
            pl.BlockSpec((1, 512, 136), lambda b, j: (b, j, 0)),
            pl.BlockSpec((136, C), lambda b, j: (0, 0)),
        ],
        out_specs=[
            pl.BlockSpec((1, 512, N1), lambda b, j: (b, j, 0)),
            pl.BlockSpec((1, 512, C), lambda b, j: (b, j, 0)),
        ],
        out_shape=[
            jax.ShapeDtypeStruct((B, N0, N1), jnp.float32),
            jax.ShapeDtypeStruct((B, N0, C), jnp.float32),
        ],
    )(pos8, nposT, f_pad, w10p)


def _d2_body(pos8_ref, nposT_ref, feat1_ref, wres1_ref, bres1_ref, w1p_ref,
             b10_ref, w2p_ref, b20_ref, d_ref, c1_ref, res1_ref, np2_ref, c2_ref):
    jb = pl.program_id(1)
    p = pos8_ref[0]
    nT = nposT_ref[0]
    d = jnp.zeros(d_ref.shape[1:], jnp.float32)
    for c in range(3):
        diff = p[:, c:c + 1] - nT[c:c + 1, :]
        d = d + diff * diff
    d_ref[0] = d
    c1_ref[0] = b10_ref[...] - jnp.dot(p, w1p_ref[...], preferred_element_type=jnp.float32, precision=lax.Precision.HIGHEST)
    res1_ref[0] = jnp.dot(feat1_ref[0], wres1_ref[...], preferred_element_type=jnp.float32, precision=lax.Precision.HIGHEST) + bres1_ref[...]
    np2 = jnp.dot(p, w2p_ref[...], preferred_element_type=jnp.float32, precision=lax.Precision.HIGHEST)
    np2_ref[0] = np2

    @pl.when(jb == 0)
    def _():
        c2_ref[0] = b20_ref[...] - np2


def _call_d2(pos2_8, npos2T, feat1, wres1, bres1, w1p8, b10, w2p8, b20):
    return pl.pallas_call(
        _d2_body,
        grid=(B, 2),
        in_specs=[
            pl.BlockSpec((1, 1024, 8), lambda b, j: (b, j, 0)),
            pl.BlockSpec((1, 8, N2), lambda b, j: (b, 0, 0)),
            pl.BlockSpec((1, 1024, C), lambda b, j: (b, j, 0)),
            pl.BlockSpec((C, C), lambda b, j: (0, 0)),
            pl.BlockSpec((1, C), lambda b, j: (0, 0)),
            pl.BlockSpec((8, C), lambda b, j: (0, 0)),
            pl.BlockSpec((1, C), lambda b, j: (0, 0)),
            pl.BlockSpec((8, C), lambda b, j: (0, 0)),
            pl.BlockSpec((1, C), lambda b, j: (0, 0)),
        ],
        out_specs=[
            pl.BlockSpec((1, 1024, N2), lambda b, j: (b, j, 0)),
            pl.BlockSpec((1, 1024, C), lambda b, j: (b, j, 0)),
            pl.BlockSpec((1, 1024, C), lambda b, j: (b, j, 0)),
            pl.BlockSpec((1, 1024, C), lambda b, j: (b, j, 0)),
            pl.BlockSpec((1, 1024, C), lambda b, j: (b, 0, 0)),
        ],
        out_shape=[
            jax.ShapeDtypeStruct((B, N1, N2), jnp.float32),
            jax.ShapeDtypeStruct((B, N1, C), jnp.float32),
            jax.ShapeDtypeStruct((B, N1, C), jnp.float32),
            jax.ShapeDtypeStruct((B, N1, C), jnp.float32),
            jax.ShapeDtypeStruct((B, N2, C), jnp.float32),
        ],
    )(pos2_8, npos2T, feat1, wres1, bres1, w1p8, b10, w2p8, b20)


def _mlp_body(g_ref, c_ref, w_ref, b_ref, lng_ref, lnb_ref, out_ref):
    cb = c_ref[0]
    w = w_ref[...]
    bb = b_ref[...]
    acc = jnp.zeros(out_ref.shape[1:], jnp.float32)
    for k in range(K):
        hk = jnp.maximum(g_ref[0][:, k * C:(k + 1) * C] + cb, 0.0)
        acc = jnp.maximum(acc, jnp.maximum(jnp.dot(hk, w, preferred_element_type=jnp.float32, precision=lax.Precision.HIGHEST) + bb, 0.0))
    mu = jnp.mean(acc, axis=1, keepdims=True)
    xc = acc - mu
    var = jnp.mean(xc * xc, axis=1, keepdims=True)
    ln = xc * lax.rsqrt(var + 1e-5) * lng_ref[...] + lnb_ref[...]
    out_ref[0] = ln * jax.nn.sigmoid(ln)


def _call_mlp(g, c, w, b, lng, lnb, n_rows, blk):
    return pl.pallas_call(
        _mlp_body,
        grid=(B, n_rows // blk),
        in_specs=[
            pl.BlockSpec((1, blk, K * C), lambda b_, i: (b_, i, 0)),
            pl.BlockSpec((1, blk, C), lambda b_, i: (b_, i, 0)),
            pl.BlockSpec((C, C), lambda b_, i: (0, 0)),
            pl.BlockSpec((1, C), lambda b_, i: (0, 0)),
            pl.BlockSpec((1, C), lambda b_, i: (0, 0)),
            pl.BlockSpec((1, C), lambda b_, i: (0, 0)),
        ],
        out_specs=pl.BlockSpec((1, blk, C), lambda b_, i: (b_, i, 0)),
        out_shape=jax.ShapeDtypeStruct((B, n_rows, C), jnp.float32),
    )(g, c, w, b, lng, lnb)


def _b2_body(och_ref, np2_ref, w2f_ref, wres2_ref, bres2_ref, res1_ref,
             z2_ref, res2_ref):
    och = och_ref[0]
    z2_ref[0] = jnp.dot(och, w2f_ref[...], preferred_element_type=jnp.float32, precision=lax.Precision.HIGHEST) + np2_ref[0]
    res2_ref[0] = (jnp.dot(och[:N2], wres2_ref[...], preferred_element_type=jnp.float32, precision=lax.Precision.HIGHEST)
                   + bres2_ref[...] + res1_ref[0])


def _call_b2(och, np2, w2f, wres2, bres2, res1):
    return pl.pallas_call(
        _b2_body,
        grid=(B,),
        in_specs=[
            pl.BlockSpec((1, N1, C), lambda b: (b, 0, 0)),
            pl.BlockSpec((1, N1, C), lambda b: (b, 0, 0)),
            pl.BlockSpec((C, C), lambda b: (0, 0)),
            pl.BlockSpec((C, C), lambda b: (0, 0)),
            pl.BlockSpec((1, C), lambda b: (0, 0)),
            pl.BlockSpec((1, N2, C), lambda b: (b, 0, 0)),
        ],
        out_specs=[
            pl.BlockSpec((1, N1, C), lambda b: (b, 0, 0)),
            pl.BlockSpec((1, N2, C), lambda b: (b, 0, 0)),
        ],
        out_shape=[
            jax.ShapeDtypeStruct((B, N1, C), jnp.float32),
            jax.ShapeDtypeStruct((B, N2, C), jnp.float32),
        ],
    )(och, np2, w2f, wres2, bres2, res1)


def _c_body(g_ref, c_ref, w_ref, b_ref, lng_ref, lnb_ref, res2_ref, out_ref):
    cb = c_ref[0]
    w = w_ref[...]
    bb = b_ref[...]
    acc = jnp.zeros(out_ref.shape[1:], jnp.float32)
    for k in range(K):
        hk = jnp.maximum(g_ref[0][:, k * C:(k + 1) * C] + cb, 0.0)
        acc = jnp.maximum(acc, jnp.maximum(jnp.dot(hk, w, preferred_element_type=jnp.float32, precision=lax.Precision.HIGHEST) + bb, 0.0))
    mu = jnp.mean(acc, axis=1, keepdims=True)
    xc = acc - mu
    var = jnp.mean(xc * xc, axis=1, keepdims=True)
    ln = xc * lax.rsqrt(var + 1e-5) * lng_ref[...] + lnb_ref[...]
    out_ref[0] = ln * jax.nn.sigmoid(ln) + res2_ref[0]


def _call_c(g2, c2, w21, b21, lng, lnb, res2):
    return pl.pallas_call(
        _c_body,
        grid=(B, 2),
        in_specs=[
            pl.BlockSpec((1, 512, K * C), lambda b_, i: (b_, i, 0)),
            pl.BlockSpec((1, 512, C), lambda b_, i: (b_, i, 0)),
            pl.BlockSpec((C, C), lambda b_, i: (0, 0)),
            pl.BlockSpec((1, C), lambda b_, i: (0, 0)),
            pl.BlockSpec((1, C), lambda b_, i: (0, 0)),
            pl.BlockSpec((1, C), lambda b_, i: (0, 0)),
            pl.BlockSpec((1, 512, C), lambda b_, i: (b_, i, 0)),
        ],
        out_specs=pl.BlockSpec((1, 512, C), lambda b_, i: (b_, i, 0)),
        out_shape=jax.ShapeDtypeStruct((B, N2, C), jnp.float32),
    )(g2, c2, w21, b21, lng, lnb, res2)


def _topk_idx(d_t, n_cand, n_ctr, row_off):
    d = jnp.transpose(d_t, (0, 2, 1))
    _, idx = lax.top_k(-d, K)
    off = (jnp.arange(B, dtype=jnp.int32) * row_off)[:, None, None]
    return (idx.astype(jnp.int32) + off).reshape(-1)


def kernel(position_matrix, channel_matrix, n_select_0, n_select_1, n_select_2, W1_0, b1_0, W1_1, b1_1, Wres1, bres1, ln1_g, ln1_b, W2_0, b2_0, W2_1, b2_1, Wres2, bres2, ln2_g, ln2_b):
    pos = position_matrix
    feat = channel_matrix

    pos8 = jnp.pad(pos, ((0, 0), (0, 0), (0, 5)))
    npos1T = jnp.transpose(pos8[:, :N1], (0, 2, 1))
    pos2_8 = pos8[:, :N1]
    npos2T = jnp.transpose(pos8[:, :N2], (0, 2, 1))
    f_pad = jnp.concatenate(
        [feat, pos, jnp.zeros((B, N0, 5), jnp.float32)], axis=-1)
    w10p = jnp.pad(W1_0, ((0, 5), (0, 0)))
    w1p8 = jnp.pad(W1_0[C:], ((0, 5), (0, 0)))
    w2p8 = jnp.pad(W2_0[C:], ((0, 5), (0, 0)))
    w2f = W2_0[:C]
    r1 = lambda v: v.reshape(1, C)
    feat1 = feat[:, :N1]

    d1_t, z1 = _call_d1(pos8, npos1T, f_pad, w10p)
    d2_t, c1, res1, np2, c2 = _call_d2(
        pos2_8, npos2T, feat1, Wres1, r1(bres1), w1p8, r1(b1_0), w2p8, r1(b2_0))

    idx1 = _topk_idx(d1_t, N0, N1, N0)
    g1 = jnp.take(z1.reshape(B * N0, C), idx1, axis=0).reshape(B, N1, K * C)

    och = _call_mlp(g1, c1, W1_1, r1(b1_1), r1(ln1_g), r1(ln1_b), N1, 512)

    z2, res2 = _call_b2(och, np2, w2f, Wres2, r1(bres2), res1)

    idx2 = _topk_idx(d2_t, N1, N2, N1)
    g2 = jnp.take(z2.reshape(B * N1, C), idx2, axis=0).reshape(B, N2, K * C)

    out_ch = _call_c(g2, c2, W2_1, r1(b2_1), r1(ln2_g), r1(ln2_b), res2)
    return (pos[:, :N2], out_ch)


# speedup vs baseline: 1.0640x; 1.0640x over previous
import functools
import jax
import jax.numpy as jnp
from jax import lax
from jax.experimental import pallas as pl
from jax.experimental.pallas import tpu as pltpu
from jax.experimental.pallas import tpu_sc as plsc

B = 4
N0 = 4096
N1 = 2048
N2 = 1024
C = 128
K = 16


def _d1_body(pos8_ref, nposT_ref, f_ref, w10_ref, d_ref, z_ref):
    p = pos8_ref[0]
    nT = nposT_ref[0]
    d = jnp.zeros(d_ref.shape[1:], jnp.float32)
    for c in range(3):
        diff = p[:, c:c + 1] - nT[c:c + 1, :]
        d = d + diff * diff
    d_ref[0] = d
    z_ref[0] = jnp.dot(f_ref[0], w10_ref[...], preferred_element_type=jnp.float32, precision=lax.Precision.HIGHEST)


def _call_d1(pos8, nposT, f_pad, w10p):
    return pl.pallas_call(
        _d1_body,
        grid=(B, N0 // 512),
        in_specs=[
            pl.BlockSpec((1, 512, 8), lambda b, j: (b, j, 0)),
            pl.BlockSpec((1, 8, N1), lambda b, j: (b, 0, 0)),
            pl.BlockSpec((1, 512, 136), lambda b, j: (b, j, 0)),
            pl.BlockSpec((136, C), lambda b, j: (0, 0)),
        ],
        out_specs=[
            pl.BlockSpec((1, 512, N1), lambda b, j: (b, j, 0)),
            pl.BlockSpec((1, 512, C), lambda b, j: (b, j, 0)),
        ],
        out_shape=[
            jax.ShapeDtypeStruct((B, N0, N1), jnp.float32),
            jax.ShapeDtypeStruct((B, N0, C), jnp.float32),
        ],
    )(pos8, nposT, f_pad, w10p)


def _d2_body(pos8_ref, nposT_ref, feat1_ref, wres1_ref, bres1_ref, w1p_ref,
             b10_ref, w2p_ref, b20_ref, d_ref, c1_ref, res1_ref, np2_ref, c2_ref):
    jb = pl.program_id(1)
    p = pos8_ref[0]
    nT = nposT_ref[0]
    d = jnp.zeros(d_ref.shape[1:], jnp.float32)
    for c in range(3):
        diff = p[:, c:c + 1] - nT[c:c + 1, :]
        d = d + diff * diff
    d_ref[0] = d
    c1_ref[0] = b10_ref[...] - jnp.dot(p, w1p_ref[...], preferred_element_type=jnp.float32, precision=lax.Precision.HIGHEST)
    res1_ref[0] = jnp.dot(feat1_ref[0], wres1_ref[...], preferred_element_type=jnp.float32, precision=lax.Precision.HIGHEST) + bres1_ref[...]
    np2 = jnp.dot(p, w2p_ref[...], preferred_element_type=jnp.float32, precision=lax.Precision.HIGHEST)
    np2_ref[0] = np2

    @pl.when(jb == 0)
    def _():
        c2_ref[0] = b20_ref[...] - np2


def _call_d2(pos2_8, npos2T, feat1, wres1, bres1, w1p8, b10, w2p8, b20):
    return pl.pallas_call(
        _d2_body,
        grid=(B, 2),
        in_specs=[
            pl.BlockSpec((1, 1024, 8), lambda b, j: (b, j, 0)),
            pl.BlockSpec((1, 8, N2), lambda b, j: (b, 0, 0)),
            pl.BlockSpec((1, 1024, C), lambda b, j: (b, j, 0)),
            pl.BlockSpec((C, C), lambda b, j: (0, 0)),
            pl.BlockSpec((1, C), lambda b, j: (0, 0)),
            pl.BlockSpec((8, C), lambda b, j: (0, 0)),
            pl.BlockSpec((1, C), lambda b, j: (0, 0)),
            pl.BlockSpec((8, C), lambda b, j: (0, 0)),
            pl.BlockSpec((1, C), lambda b, j: (0, 0)),
        ],
        out_specs=[
            pl.BlockSpec((1, 1024, N2), lambda b, j: (b, j, 0)),
            pl.BlockSpec((1, 1024, C), lambda b, j: (b, j, 0)),
            pl.BlockSpec((1, 1024, C), lambda b, j: (b, j, 0)),
            pl.BlockSpec((1, 1024, C), lambda b, j: (b, j, 0)),
            pl.BlockSpec((1, 1024, C), lambda b, j: (b, 0, 0)),
        ],
        out_shape=[
            jax.ShapeDtypeStruct((B, N1, N2), jnp.float32),
            jax.ShapeDtypeStruct((B, N1, C), jnp.float32),
            jax.ShapeDtypeStruct((B, N1, C), jnp.float32),
            jax.ShapeDtypeStruct((B, N1, C), jnp.float32),
            jax.ShapeDtypeStruct((B, N2, C), jnp.float32),
        ],
    )(pos2_8, npos2T, feat1, wres1, bres1, w1p8, b10, w2p8, b20)


def _mlp_body(g_ref, c_ref, w_ref, b_ref, lng_ref, lnb_ref, out_ref):
    cb = c_ref[0]
    w = w_ref[...]
    bb = b_ref[...]
    acc = jnp.zeros(out_ref.shape[1:], jnp.float32)
    for k in range(K):
        hk = jnp.maximum(g_ref[0][:, k * C:(k + 1) * C] + cb, 0.0)
        acc = jnp.maximum(acc, jnp.maximum(jnp.dot(hk, w, preferred_element_type=jnp.float32, precision=lax.Precision.HIGHEST) + bb, 0.0))
    mu = jnp.mean(acc, axis=1, keepdims=True)
    xc = acc - mu
    var = jnp.mean(xc * xc, axis=1, keepdims=True)
    ln = xc * lax.rsqrt(var + 1e-5) * lng_ref[...] + lnb_ref[...]
    out_ref[0] = ln * jax.nn.sigmoid(ln)


def _call_mlp(g, c, w, b, lng, lnb, n_rows, blk):
    return pl.pallas_call(
        _mlp_body,
        grid=(B, n_rows // blk),
        in_specs=[
            pl.BlockSpec((1, blk, K * C), lambda b_, i: (b_, i, 0)),
            pl.BlockSpec((1, blk, C), lambda b_, i: (b_, i, 0)),
            pl.BlockSpec((C, C), lambda b_, i: (0, 0)),
            pl.BlockSpec((1, C), lambda b_, i: (0, 0)),
            pl.BlockSpec((1, C), lambda b_, i: (0, 0)),
            pl.BlockSpec((1, C), lambda b_, i: (0, 0)),
        ],
        out_specs=pl.BlockSpec((1, blk, C), lambda b_, i: (b_, i, 0)),
        out_shape=jax.ShapeDtypeStruct((B, n_rows, C), jnp.float32),
    )(g, c, w, b, lng, lnb)


def _b2_body(och_ref, np2_ref, w2f_ref, wres2_ref, bres2_ref, res1_ref,
             z2_ref, res2_ref):
    och = och_ref[0]
    z2_ref[0] = jnp.dot(och, w2f_ref[...], preferred_element_type=jnp.float32, precision=lax.Precision.HIGHEST) + np2_ref[0]
    res2_ref[0] = (jnp.dot(och[:N2], wres2_ref[...], preferred_element_type=jnp.float32, precision=lax.Precision.HIGHEST)
                   + bres2_ref[...] + res1_ref[0])


def _call_b2(och, np2, w2f, wres2, bres2, res1):
    return pl.pallas_call(
        _b2_body,
        grid=(B,),
        in_specs=[
            pl.BlockSpec((1, N1, C), lambda b: (b, 0, 0)),
            pl.BlockSpec((1, N1, C), lambda b: (b, 0, 0)),
            pl.BlockSpec((C, C), lambda b: (0, 0)),
            pl.BlockSpec((C, C), lambda b: (0, 0)),
            pl.BlockSpec((1, C), lambda b: (0, 0)),
            pl.BlockSpec((1, N2, C), lambda b: (b, 0, 0)),
        ],
        out_specs=[
            pl.BlockSpec((1, N1, C), lambda b: (b, 0, 0)),
            pl.BlockSpec((1, N2, C), lambda b: (b, 0, 0)),
        ],
        out_shape=[
            jax.ShapeDtypeStruct((B, N1, C), jnp.float32),
            jax.ShapeDtypeStruct((B, N2, C), jnp.float32),
        ],
    )(och, np2, w2f, wres2, bres2, res1)


def _c_body(g_ref, c_ref, w_ref, b_ref, lng_ref, lnb_ref, res2_ref, out_ref):
    cb = c_ref[0]
    w = w_ref[...]
    bb = b_ref[...]
    acc = jnp.zeros(out_ref.shape[1:], jnp.float32)
    for k in range(K):
        hk = jnp.maximum(g_ref[0][:, k * C:(k + 1) * C] + cb, 0.0)
        acc = jnp.maximum(acc, jnp.maximum(jnp.dot(hk, w, preferred_element_type=jnp.float32, precision=lax.Precision.HIGHEST) + bb, 0.0))
    mu = jnp.mean(acc, axis=1, keepdims=True)
    xc = acc - mu
    var = jnp.mean(xc * xc, axis=1, keepdims=True)
    ln = xc * lax.rsqrt(var + 1e-5) * lng_ref[...] + lnb_ref[...]
    out_ref[0] = ln * jax.nn.sigmoid(ln) + res2_ref[0]


def _call_c(g2, c2, w21, b21, lng, lnb, res2):
    return pl.pallas_call(
        _c_body,
        grid=(B, 2),
        in_specs=[
            pl.BlockSpec((1, 512, K * C), lambda b_, i: (b_, i, 0)),
            pl.BlockSpec((1, 512, C), lambda b_, i: (b_, i, 0)),
            pl.BlockSpec((C, C), lambda b_, i: (0, 0)),
            pl.BlockSpec((1, C), lambda b_, i: (0, 0)),
            pl.BlockSpec((1, C), lambda b_, i: (0, 0)),
            pl.BlockSpec((1, C), lambda b_, i: (0, 0)),
            pl.BlockSpec((1, 512, C), lambda b_, i: (b_, i, 0)),
        ],
        out_specs=pl.BlockSpec((1, 512, C), lambda b_, i: (b_, i, 0)),
        out_shape=jax.ShapeDtypeStruct((B, N2, C), jnp.float32),
    )(g2, c2, w21, b21, lng, lnb, res2)


_SC_MESH = lambda: plsc.VectorSubcoreMesh(core_axis_name="c", subcore_axis_name="s")
_NW = 32
_GCH = 128


def _sc_gather(table, idx):
    M = idx.shape[0]
    m_per_w = M // _NW
    nch = m_per_w // _GCH

    @functools.partial(
        pl.kernel,
        mesh=_SC_MESH(),
        out_type=jax.ShapeDtypeStruct((M, C), jnp.float32),
        scratch_types=[
            pltpu.VMEM((_GCH,), jnp.int32),
            pltpu.VMEM((_GCH, C), jnp.float32),
            pltpu.SemaphoreType.DMA,
        ],
    )
    def k(table_hbm, idx_hbm, out_hbm, idx_v, rows_v, sem):
        wid = lax.axis_index("s") * 2 + lax.axis_index("c")
        base = wid * m_per_w

        def chunk(i, carry):
            off = base + i * _GCH
            pltpu.sync_copy(idx_hbm.at[pl.ds(off, _GCH)], idx_v)
            pltpu.async_copy(table_hbm.at[idx_v], rows_v, sem).wait()
            pltpu.sync_copy(rows_v, out_hbm.at[pl.ds(off, _GCH)])
            return carry

        lax.fori_loop(0, nch, chunk, 0)

    return k(table, idx)


def _topk_idx(d_t, n_cand, n_ctr, row_off):
    d = jnp.transpose(d_t, (0, 2, 1))
    _, idx = lax.top_k(-d, K)
    off = (jnp.arange(B, dtype=jnp.int32) * row_off)[:, None, None]
    return (idx.astype(jnp.int32) + off).reshape(-1)


def kernel(position_matrix, channel_matrix, n_select_0, n_select_1, n_select_2, W1_0, b1_0, W1_1, b1_1, Wres1, bres1, ln1_g, ln1_b, W2_0, b2_0, W2_1, b2_1, Wres2, bres2, ln2_g, ln2_b):
    pos = position_matrix
    feat = channel_matrix

    pos8 = jnp.pad(pos, ((0, 0), (0, 0), (0, 5)))
    npos1T = jnp.transpose(pos8[:, :N1], (0, 2, 1))
    pos2_8 = pos8[:, :N1]
    npos2T = jnp.transpose(pos8[:, :N2], (0, 2, 1))
    f_pad = jnp.concatenate(
        [feat, pos, jnp.zeros((B, N0, 5), jnp.float32)], axis=-1)
    w10p = jnp.pad(W1_0, ((0, 5), (0, 0)))
    w1p8 = jnp.pad(W1_0[C:], ((0, 5), (0, 0)))
    w2p8 = jnp.pad(W2_0[C:], ((0, 5), (0, 0)))
    w2f = W2_0[:C]
    r1 = lambda v: v.reshape(1, C)
    feat1 = feat[:, :N1]

    d1_t, z1 = _call_d1(pos8, npos1T, f_pad, w10p)
    d2_t, c1, res1, np2, c2 = _call_d2(
        pos2_8, npos2T, feat1, Wres1, r1(bres1), w1p8, r1(b1_0), w2p8, r1(b2_0))

    idx1 = _topk_idx(d1_t, N0, N1, N0)
    g1 = _sc_gather(z1.reshape(B * N0, C), idx1).reshape(B, N1, K * C)

    och = _call_mlp(g1, c1, W1_1, r1(b1_1), r1(ln1_g), r1(ln1_b), N1, 512)

    z2, res2 = _call_b2(och, np2, w2f, Wres2, r1(bres2), res1)

    idx2 = _topk_idx(d2_t, N1, N2, N1)
    g2 = _sc_gather(z2.reshape(B * N1, C), idx2).reshape(B, N2, K * C)

    out_ch = _call_c(g2, c2, W2_1, r1(b2_1), r1(ln2_g), r1(ln2_b), res2)
    return (pos[:, :N2], out_ch)


# speedup vs baseline: 3.7882x; 3.5603x over previous
import functools
import jax
import jax.numpy as jnp
from jax import lax
from jax.experimental import pallas as pl
from jax.experimental.pallas import tpu as pltpu
from jax.experimental.pallas import tpu_sc as plsc

B = 4
N0 = 4096
N1 = 2048
N2 = 1024
C = 128
K = 16


def _z_body(f_ref, w10_ref, z_ref):
    z_ref[0] = jnp.dot(f_ref[0], w10_ref[...], preferred_element_type=jnp.float32, precision=lax.Precision.HIGHEST)


def _call_z(f_pad, w10p):
    return pl.pallas_call(
        _z_body,
        grid=(B, N0 // 512),
        in_specs=[
            pl.BlockSpec((1, 512, 136), lambda b, j: (b, j, 0)),
            pl.BlockSpec((136, C), lambda b, j: (0, 0)),
        ],
        out_specs=pl.BlockSpec((1, 512, C), lambda b, j: (b, j, 0)),
        out_shape=jax.ShapeDtypeStruct((B, N0, C), jnp.float32),
    )(f_pad, w10p)


def _d_body(ctr_ref, candT_ref, d_ref):
    p = ctr_ref[0]
    nT = candT_ref[0]
    d = jnp.zeros(d_ref.shape[1:], jnp.float32)
    for c in range(3):
        diff = p[:, c:c + 1] - nT[c:c + 1, :]
        d = d + diff * diff
    d_ref[0] = d


def _call_d(ctr8, candT8, nctr, ncand):
    return pl.pallas_call(
        _d_body,
        grid=(B, nctr // 512),
        in_specs=[
            pl.BlockSpec((1, 512, 8), lambda b, j: (b, j, 0)),
            pl.BlockSpec((1, 8, ncand), lambda b, j: (b, 0, 0)),
        ],
        out_specs=pl.BlockSpec((1, 512, ncand), lambda b, j: (b, j, 0)),
        out_shape=jax.ShapeDtypeStruct((B, nctr, ncand), jnp.float32),
    )(ctr8, candT8)


def _p_body(pos8_ref, feat1_ref, wres1_ref, bres1_ref, w1p_ref,
            b10_ref, w2p_ref, b20_ref, c1_ref, res1_ref, np2_ref, c2_ref):
    jb = pl.program_id(1)
    p = pos8_ref[0]
    c1_ref[0] = b10_ref[...] - jnp.dot(p, w1p_ref[...], preferred_element_type=jnp.float32, precision=lax.Precision.HIGHEST)
    res1_ref[0] = jnp.dot(feat1_ref[0], wres1_ref[...], preferred_element_type=jnp.float32, precision=lax.Precision.HIGHEST) + bres1_ref[...]
    np2 = jnp.dot(p, w2p_ref[...], preferred_element_type=jnp.float32, precision=lax.Precision.HIGHEST)
    np2_ref[0] = np2

    @pl.when(jb == 0)
    def _():
        c2_ref[0] = b20_ref[...] - np2


def _call_p(pos2_8, feat1, wres1, bres1, w1p8, b10, w2p8, b20):
    return pl.pallas_call(
        _p_body,
        grid=(B, 2),
        in_specs=[
            pl.BlockSpec((1, 1024, 8), lambda b, j: (b, j, 0)),
            pl.BlockSpec((1, 1024, C), lambda b, j: (b, j, 0)),
            pl.BlockSpec((C, C), lambda b, j: (0, 0)),
            pl.BlockSpec((1, C), lambda b, j: (0, 0)),
            pl.BlockSpec((8, C), lambda b, j: (0, 0)),
            pl.BlockSpec((1, C), lambda b, j: (0, 0)),
            pl.BlockSpec((8, C), lambda b, j: (0, 0)),
            pl.BlockSpec((1, C), lambda b, j: (0, 0)),
        ],
        out_specs=[
            pl.BlockSpec((1, 1024, C), lambda b, j: (b, j, 0)),
            pl.BlockSpec((1, 1024, C), lambda b, j: (b, j, 0)),
            pl.BlockSpec((1, 1024, C), lambda b, j: (b, j, 0)),
            pl.BlockSpec((1, 1024, C), lambda b, j: (b, 0, 0)),
        ],
        out_shape=[
            jax.ShapeDtypeStruct((B, N1, C), jnp.float32),
            jax.ShapeDtypeStruct((B, N1, C), jnp.float32),
            jax.ShapeDtypeStruct((B, N1, C), jnp.float32),
            jax.ShapeDtypeStruct((B, N2, C), jnp.float32),
        ],
    )(pos2_8, feat1, wres1, bres1, w1p8, b10, w2p8, b20)


def _mlp_body(g_ref, c_ref, w_ref, b_ref, lng_ref, lnb_ref, out_ref):
    cb = c_ref[0]
    w = w_ref[...]
    bb = b_ref[...]
    acc = jnp.zeros(out_ref.shape[1:], jnp.float32)
    for k in range(K):
        hk = jnp.maximum(g_ref[0][:, k * C:(k + 1) * C] + cb, 0.0)
        acc = jnp.maximum(acc, jnp.maximum(jnp.dot(hk, w, preferred_element_type=jnp.float32, precision=lax.Precision.HIGHEST) + bb, 0.0))
    mu = jnp.mean(acc, axis=1, keepdims=True)
    xc = acc - mu
    var = jnp.mean(xc * xc, axis=1, keepdims=True)
    ln = xc * lax.rsqrt(var + 1e-5) * lng_ref[...] + lnb_ref[...]
    out_ref[0] = ln * jax.nn.sigmoid(ln)


def _call_mlp(g, c, w, b, lng, lnb, n_rows, blk):
    return pl.pallas_call(
        _mlp_body,
        grid=(B, n_rows // blk),
        in_specs=[
            pl.BlockSpec((1, blk, K * C), lambda b_, i: (b_, i, 0)),
            pl.BlockSpec((1, blk, C), lambda b_, i: (b_, i, 0)),
            pl.BlockSpec((C, C), lambda b_, i: (0, 0)),
            pl.BlockSpec((1, C), lambda b_, i: (0, 0)),
            pl.BlockSpec((1, C), lambda b_, i: (0, 0)),
            pl.BlockSpec((1, C), lambda b_, i: (0, 0)),
        ],
        out_specs=pl.BlockSpec((1, blk, C), lambda b_, i: (b_, i, 0)),
        out_shape=jax.ShapeDtypeStruct((B, n_rows, C), jnp.float32),
    )(g, c, w, b, lng, lnb)


def _b2_body(och_ref, np2_ref, w2f_ref, wres2_ref, bres2_ref, res1_ref,
             z2_ref, res2_ref):
    och = och_ref[0]
    z2_ref[0] = jnp.dot(och, w2f_ref[...], preferred_element_type=jnp.float32, precision=lax.Precision.HIGHEST) + np2_ref[0]
    res2_ref[0] = (jnp.dot(och[:N2], wres2_ref[...], preferred_element_type=jnp.float32, precision=lax.Precision.HIGHEST)
                   + bres2_ref[...] + res1_ref[0])


def _call_b2(och, np2, w2f, wres2, bres2, res1):
    return pl.pallas_call(
        _b2_body,
        grid=(B,),
        in_specs=[
            pl.BlockSpec((1, N1, C), lambda b: (b, 0, 0)),
            pl.BlockSpec((1, N1, C), lambda b: (b, 0, 0)),
            pl.BlockSpec((C, C), lambda b: (0, 0)),
            pl.BlockSpec((C, C), lambda b: (0, 0)),
            pl.BlockSpec((1, C), lambda b: (0, 0)),
            pl.BlockSpec((1, N2, C), lambda b: (b, 0, 0)),
        ],
        out_specs=[
            pl.BlockSpec((1, N1, C), lambda b: (b, 0, 0)),
            pl.BlockSpec((1, N2, C), lambda b: (b, 0, 0)),
        ],
        out_shape=[
            jax.ShapeDtypeStruct((B, N1, C), jnp.float32),
            jax.ShapeDtypeStruct((B, N2, C), jnp.float32),
        ],
    )(och, np2, w2f, wres2, bres2, res1)


def _c_body(g_ref, c_ref, w_ref, b_ref, lng_ref, lnb_ref, res2_ref, out_ref):
    cb = c_ref[0]
    w = w_ref[...]
    bb = b_ref[...]
    acc = jnp.zeros(out_ref.shape[1:], jnp.float32)
    for k in range(K):
        hk = jnp.maximum(g_ref[0][:, k * C:(k + 1) * C] + cb, 0.0)
        acc = jnp.maximum(acc, jnp.maximum(jnp.dot(hk, w, preferred_element_type=jnp.float32, precision=lax.Precision.HIGHEST) + bb, 0.0))
    mu = jnp.mean(acc, axis=1, keepdims=True)
    xc = acc - mu
    var = jnp.mean(xc * xc, axis=1, keepdims=True)
    ln = xc * lax.rsqrt(var + 1e-5) * lng_ref[...] + lnb_ref[...]
    out_ref[0] = ln * jax.nn.sigmoid(ln) + res2_ref[0]


def _call_c(g2, c2, w21, b21, lng, lnb, res2):
    return pl.pallas_call(
        _c_body,
        grid=(B, 2),
        in_specs=[
            pl.BlockSpec((1, 512, K * C), lambda b_, i: (b_, i, 0)),
            pl.BlockSpec((1, 512, C), lambda b_, i: (b_, i, 0)),
            pl.BlockSpec((C, C), lambda b_, i: (0, 0)),
            pl.BlockSpec((1, C), lambda b_, i: (0, 0)),
            pl.BlockSpec((1, C), lambda b_, i: (0, 0)),
            pl.BlockSpec((1, C), lambda b_, i: (0, 0)),
            pl.BlockSpec((1, 512, C), lambda b_, i: (b_, i, 0)),
        ],
        out_specs=pl.BlockSpec((1, 512, C), lambda b_, i: (b_, i, 0)),
        out_shape=jax.ShapeDtypeStruct((B, N2, C), jnp.float32),
    )(g2, c2, w21, b21, lng, lnb, res2)


_SC_MESH = lambda: plsc.VectorSubcoreMesh(core_axis_name="c", subcore_axis_name="s")
_NW = 32
_GCH = 128


def _sc_gather(table, idx):
    M = idx.shape[0]
    m_per_w = M // _NW
    nch = m_per_w // _GCH

    @functools.partial(
        pl.kernel,
        mesh=_SC_MESH(),
        out_type=jax.ShapeDtypeStruct((M, C), jnp.float32),
        scratch_types=[
            pltpu.VMEM((_GCH,), jnp.int32),
            pltpu.VMEM((_GCH, C), jnp.float32),
            pltpu.SemaphoreType.DMA,
        ],
    )
    def k(table_hbm, idx_hbm, out_hbm, idx_v, rows_v, sem):
        wid = lax.axis_index("s") * 2 + lax.axis_index("c")
        base = wid * m_per_w

        def chunk(i, carry):
            off = base + i * _GCH
            pltpu.sync_copy(idx_hbm.at[pl.ds(off, _GCH)], idx_v)
            pltpu.async_copy(table_hbm.at[idx_v], rows_v, sem).wait()
            pltpu.sync_copy(rows_v, out_hbm.at[pl.ds(off, _GCH)])
            return carry

        lax.fori_loop(0, nch, chunk, 0)

    return k(table, idx)


_CAP = 512


def _sc_topk(d, nctr, ncand):
    gpb = nctr // 16
    groups = B * gpb
    gpw = groups // _NW
    nsv = ncand // 256
    gshift = {128: 7, 64: 6}[gpb]

    @functools.partial(
        pl.kernel,
        mesh=_SC_MESH(),
        compiler_params=pltpu.CompilerParams(needs_layout_passes=False),
        out_type=jax.ShapeDtypeStruct((B * nctr, K), jnp.int32),
        scratch_types=[
            pltpu.VMEM((16, ncand), jnp.float32),
            pltpu.VMEM((_CAP + 16,), jnp.float32),
            pltpu.VMEM((_CAP + 16,), jnp.int32),
            pltpu.VMEM((16, K), jnp.int32),
        ],
    )
    def k(d_hbm, out_hbm, dbuf, vbuf, ibuf, obuf):
        wid = lax.axis_index("s") * 2 + lax.axis_index("c")
        lane = lax.iota(jnp.int32, 16)
        inf_v = jnp.full((16,), jnp.inf, jnp.float32)
        zero_i = jnp.zeros((16,), jnp.int32)

        def group_body(gi, carry):
            g = wid * gpw + gi
            b = lax.shift_right_logical(g, gshift)
            i0 = jnp.bitwise_and(g, gpb - 1) * 16
            pltpu.sync_copy(d_hbm.at[b, pl.ds(i0, 16), :], dbuf)
            joff = b * ncand

            def center_body(l, carry2):
                t = jnp.float32(-jnp.inf)
                for s in range(16):
                    def seg_body(u, m):
                        return jnp.minimum(m, dbuf[l, pl.ds((s * nsv + u) * 16, 16)])
                    m = lax.fori_loop(0, nsv, seg_body, inf_v)
                    t = jnp.maximum(t, jnp.min(m))

                def clr(i, c):
                    vbuf[pl.ds(i * 16, 16)] = inf_v
                    return c
                lax.fori_loop(0, (_CAP + 16) // 16, clr, 0)

                def scan_body(jj, fill):
                    v = dbuf[l, pl.ds(jj * 16, 16)]
                    msk = v <= t

                    def ins(f):
                        fw = jnp.minimum(f, _CAP)
                        plsc.store_compressed(vbuf.at[pl.ds(fw, 16)], v, mask=msk)
                        iv = lane + (jj * 16 + joff)
                        plsc.store_compressed(ibuf.at[pl.ds(fw, 16)], iv, mask=msk)
                        return f + jnp.sum(msk.astype(jnp.int32))

                    return lax.cond(jnp.any(msk), ins, lambda f: f, fill)

                fill = lax.fori_loop(0, ncand // 16, scan_body, jnp.int32(0))

                nch = lax.shift_right_logical(jnp.minimum(fill, _CAP) + 15, 4)

                def ch_body(cc, bvbi):
                    bv, bi = bvbi
                    cv = vbuf[pl.ds(cc * 16, 16)]
                    ci = ibuf[pl.ds(cc * 16, 16)]
                    cvs, cis = plsc.sort_key_val(cv, ci)
                    bvr = lax.rev(bv, (0,))
                    bir = lax.rev(bi, (0,))
                    take = cvs <= bvr
                    nv = jnp.where(take, cvs, bvr)
                    ni = jnp.where(take, cis, bir)
                    r = plsc.sort_key_val(nv, ni)
                    return (r[0], r[1])

                bv, bi = lax.fori_loop(0, nch, ch_body, (inf_v, zero_i))
                obuf[l] = bi
                return carry2

            lax.fori_loop(0, 16, center_body, 0)
            pltpu.sync_copy(obuf, out_hbm.at[pl.ds(g * 16, 16)])
            return carry

        lax.fori_loop(0, gpw, group_body, 0)

    return k(d)


def kernel(position_matrix, channel_matrix, n_select_0, n_select_1, n_select_2, W1_0, b1_0, W1_1, b1_1, Wres1, bres1, ln1_g, ln1_b, W2_0, b2_0, W2_1, b2_1, Wres2, bres2, ln2_g, ln2_b):
    pos = position_matrix
    feat = channel_matrix

    pos8 = jnp.pad(pos, ((0, 0), (0, 0), (0, 5)))
    pos8T = jnp.transpose(pos8, (0, 2, 1))
    pos2_8 = pos8[:, :N1]
    pos2_8T = pos8T[:, :, :N1]
    npos2_8 = pos8[:, :N2]
    f_pad = jnp.concatenate(
        [feat, pos, jnp.zeros((B, N0, 5), jnp.float32)], axis=-1)
    w10p = jnp.pad(W1_0, ((0, 5), (0, 0)))
    w1p8 = jnp.pad(W1_0[C:], ((0, 5), (0, 0)))
    w2p8 = jnp.pad(W2_0[C:], ((0, 5), (0, 0)))
    w2f = W2_0[:C]
    r1 = lambda v: v.reshape(1, C)
    feat1 = feat[:, :N1]

    z1 = _call_z(f_pad, w10p)
    d1 = _call_d(pos2_8, pos8T, N1, N0)
    d2 = _call_d(npos2_8, pos2_8T, N2, N1)
    c1, res1, np2, c2 = _call_p(
        pos2_8, feat1, Wres1, r1(bres1), w1p8, r1(b1_0), w2p8, r1(b2_0))

    idx1 = _sc_topk(d1, N1, N0).reshape(-1)
    g1 = _sc_gather(z1.reshape(B * N0, C), idx1).reshape(B, N1, K * C)

    och = _call_mlp(g1, c1, W1_1, r1(b1_1), r1(ln1_g), r1(ln1_b), N1, 512)

    z2, res2 = _call_b2(och, np2, w2f, Wres2, r1(bres2), res1)

    idx2 = _sc_topk(d2, N2, N1).reshape(-1)
    g2 = _sc_gather(z2.reshape(B * N1, C), idx2).reshape(B, N2, K * C)

    out_ch = _call_c(g2, c2, W2_1, r1(b2_1), r1(ln2_g), r1(ln2_b), res2)
    return (pos[:, :N2], out_ch)


# speedup vs baseline: 6.3521x; 1.6768x over previous
import functools
import jax
import jax.numpy as jnp
from jax import lax
from jax.experimental import pallas as pl
from jax.experimental.pallas import tpu as pltpu
from jax.experimental.pallas import tpu_sc as plsc

B = 4
N0 = 4096
N1 = 2048
N2 = 1024
C = 128
K = 16


def _z_body(f_ref, w10_ref, z_ref):
    z_ref[0] = jnp.dot(f_ref[0], w10_ref[...], preferred_element_type=jnp.float32, precision=lax.Precision.HIGHEST)


def _call_z(f_pad, w10p):
    return pl.pallas_call(
        _z_body,
        grid=(B, N0 // 512),
        in_specs=[
            pl.BlockSpec((1, 512, 136), lambda b, j: (b, j, 0)),
            pl.BlockSpec((136, C), lambda b, j: (0, 0)),
        ],
        out_specs=pl.BlockSpec((1, 512, C), lambda b, j: (b, j, 0)),
        out_shape=jax.ShapeDtypeStruct((B, N0, C), jnp.float32),
    )(f_pad, w10p)


def _d_body(ctr_ref, candT_ref, d_ref):
    p = ctr_ref[0]
    nT = candT_ref[0]
    d = jnp.zeros(d_ref.shape[1:], jnp.float32)
    for c in range(3):
        diff = p[:, c:c + 1] - nT[c:c + 1, :]
        d = d + diff * diff
    d_ref[0] = d


def _call_d(ctr8, candT8, nctr, ncand):
    return pl.pallas_call(
        _d_body,
        grid=(B, nctr // 512),
        in_specs=[
            pl.BlockSpec((1, 512, 8), lambda b, j: (b, j, 0)),
            pl.BlockSpec((1, 8, ncand), lambda b, j: (b, 0, 0)),
        ],
        out_specs=pl.BlockSpec((1, 512, ncand), lambda b, j: (b, j, 0)),
        out_shape=jax.ShapeDtypeStruct((B, nctr, ncand), jnp.float32),
    )(ctr8, candT8)


def _p_body(pos8_ref, feat1_ref, wres1_ref, bres1_ref, w1p_ref,
            b10_ref, w2p_ref, b20_ref, c1_ref, res1_ref, np2_ref, c2_ref):
    jb = pl.program_id(1)
    p = pos8_ref[0]
    c1_ref[0] = b10_ref[...] - jnp.dot(p, w1p_ref[...], preferred_element_type=jnp.float32, precision=lax.Precision.HIGHEST)
    res1_ref[0] = jnp.dot(feat1_ref[0], wres1_ref[...], preferred_element_type=jnp.float32, precision=lax.Precision.HIGHEST) + bres1_ref[...]
    np2 = jnp.dot(p, w2p_ref[...], preferred_element_type=jnp.float32, precision=lax.Precision.HIGHEST)
    np2_ref[0] = np2

    @pl.when(jb == 0)
    def _():
        c2_ref[0] = b20_ref[...] - np2


def _call_p(pos2_8, feat1, wres1, bres1, w1p8, b10, w2p8, b20):
    return pl.pallas_call(
        _p_body,
        grid=(B, 2),
        in_specs=[
            pl.BlockSpec((1, 1024, 8), lambda b, j: (b, j, 0)),
            pl.BlockSpec((1, 1024, C), lambda b, j: (b, j, 0)),
            pl.BlockSpec((C, C), lambda b, j: (0, 0)),
            pl.BlockSpec((1, C), lambda b, j: (0, 0)),
            pl.BlockSpec((8, C), lambda b, j: (0, 0)),
            pl.BlockSpec((1, C), lambda b, j: (0, 0)),
            pl.BlockSpec((8, C), lambda b, j: (0, 0)),
            pl.BlockSpec((1, C), lambda b, j: (0, 0)),
        ],
        out_specs=[
            pl.BlockSpec((1, 1024, C), lambda b, j: (b, j, 0)),
            pl.BlockSpec((1, 1024, C), lambda b, j: (b, j, 0)),
            pl.BlockSpec((1, 1024, C), lambda b, j: (b, j, 0)),
            pl.BlockSpec((1, 1024, C), lambda b, j: (b, 0, 0)),
        ],
        out_shape=[
            jax.ShapeDtypeStruct((B, N1, C), jnp.float32),
            jax.ShapeDtypeStruct((B, N1, C), jnp.float32),
            jax.ShapeDtypeStruct((B, N1, C), jnp.float32),
            jax.ShapeDtypeStruct((B, N2, C), jnp.float32),
        ],
    )(pos2_8, feat1, wres1, bres1, w1p8, b10, w2p8, b20)


def _mlp_body(g_ref, c_ref, w_ref, b_ref, lng_ref, lnb_ref, out_ref):
    cb = c_ref[0]
    w = w_ref[...]
    bb = b_ref[...]
    acc = jnp.zeros(out_ref.shape[1:], jnp.float32)
    for k in range(K):
        hk = jnp.maximum(g_ref[0][:, k * C:(k + 1) * C] + cb, 0.0)
        acc = jnp.maximum(acc, jnp.maximum(jnp.dot(hk, w, preferred_element_type=jnp.float32, precision=lax.Precision.HIGHEST) + bb, 0.0))
    mu = jnp.mean(acc, axis=1, keepdims=True)
    xc = acc - mu
    var = jnp.mean(xc * xc, axis=1, keepdims=True)
    ln = xc * lax.rsqrt(var + 1e-5) * lng_ref[...] + lnb_ref[...]
    out_ref[0] = ln * jax.nn.sigmoid(ln)


def _call_mlp(g, c, w, b, lng, lnb, n_rows, blk):
    return pl.pallas_call(
        _mlp_body,
        grid=(B, n_rows // blk),
        in_specs=[
            pl.BlockSpec((1, blk, K * C), lambda b_, i: (b_, i, 0)),
            pl.BlockSpec((1, blk, C), lambda b_, i: (b_, i, 0)),
            pl.BlockSpec((C, C), lambda b_, i: (0, 0)),
            pl.BlockSpec((1, C), lambda b_, i: (0, 0)),
            pl.BlockSpec((1, C), lambda b_, i: (0, 0)),
            pl.BlockSpec((1, C), lambda b_, i: (0, 0)),
        ],
        out_specs=pl.BlockSpec((1, blk, C), lambda b_, i: (b_, i, 0)),
        out_shape=jax.ShapeDtypeStruct((B, n_rows, C), jnp.float32),
    )(g, c, w, b, lng, lnb)


def _b2_body(och_ref, np2_ref, w2f_ref, wres2_ref, bres2_ref, res1_ref,
             z2_ref, res2_ref):
    och = och_ref[0]
    z2_ref[0] = jnp.dot(och, w2f_ref[...], preferred_element_type=jnp.float32, precision=lax.Precision.HIGHEST) + np2_ref[0]
    res2_ref[0] = (jnp.dot(och[:N2], wres2_ref[...], preferred_element_type=jnp.float32, precision=lax.Precision.HIGHEST)
                   + bres2_ref[...] + res1_ref[0])


def _call_b2(och, np2, w2f, wres2, bres2, res1):
    return pl.pallas_call(
        _b2_body,
        grid=(B,),
        in_specs=[
            pl.BlockSpec((1, N1, C), lambda b: (b, 0, 0)),
            pl.BlockSpec((1, N1, C), lambda b: (b, 0, 0)),
            pl.BlockSpec((C, C), lambda b: (0, 0)),
            pl.BlockSpec((C, C), lambda b: (0, 0)),
            pl.BlockSpec((1, C), lambda b: (0, 0)),
            pl.BlockSpec((1, N2, C), lambda b: (b, 0, 0)),
        ],
        out_specs=[
            pl.BlockSpec((1, N1, C), lambda b: (b, 0, 0)),
            pl.BlockSpec((1, N2, C), lambda b: (b, 0, 0)),
        ],
        out_shape=[
            jax.ShapeDtypeStruct((B, N1, C), jnp.float32),
            jax.ShapeDtypeStruct((B, N2, C), jnp.float32),
        ],
    )(och, np2, w2f, wres2, bres2, res1)


def _c_body(g_ref, c_ref, w_ref, b_ref, lng_ref, lnb_ref, res2_ref, out_ref):
    cb = c_ref[0]
    w = w_ref[...]
    bb = b_ref[...]
    acc = jnp.zeros(out_ref.shape[1:], jnp.float32)
    for k in range(K):
        hk = jnp.maximum(g_ref[0][:, k * C:(k + 1) * C] + cb, 0.0)
        acc = jnp.maximum(acc, jnp.maximum(jnp.dot(hk, w, preferred_element_type=jnp.float32, precision=lax.Precision.HIGHEST) + bb, 0.0))
    mu = jnp.mean(acc, axis=1, keepdims=True)
    xc = acc - mu
    var = jnp.mean(xc * xc, axis=1, keepdims=True)
    ln = xc * lax.rsqrt(var + 1e-5) * lng_ref[...] + lnb_ref[...]
    out_ref[0] = ln * jax.nn.sigmoid(ln) + res2_ref[0]


def _call_c(g2, c2, w21, b21, lng, lnb, res2):
    return pl.pallas_call(
        _c_body,
        grid=(B, 2),
        in_specs=[
            pl.BlockSpec((1, 512, K * C), lambda b_, i: (b_, i, 0)),
            pl.BlockSpec((1, 512, C), lambda b_, i: (b_, i, 0)),
            pl.BlockSpec((C, C), lambda b_, i: (0, 0)),
            pl.BlockSpec((1, C), lambda b_, i: (0, 0)),
            pl.BlockSpec((1, C), lambda b_, i: (0, 0)),
            pl.BlockSpec((1, C), lambda b_, i: (0, 0)),
            pl.BlockSpec((1, 512, C), lambda b_, i: (b_, i, 0)),
        ],
        out_specs=pl.BlockSpec((1, 512, C), lambda b_, i: (b_, i, 0)),
        out_shape=jax.ShapeDtypeStruct((B, N2, C), jnp.float32),
    )(g2, c2, w21, b21, lng, lnb, res2)


_SC_MESH = lambda: plsc.VectorSubcoreMesh(core_axis_name="c", subcore_axis_name="s")
_NW = 32
_GCH = 128


def _sc_gather(table, idx):
    M = idx.shape[0]
    m_per_w = M // _NW
    nch = m_per_w // _GCH

    @functools.partial(
        pl.kernel,
        mesh=_SC_MESH(),
        out_type=jax.ShapeDtypeStruct((M, C), jnp.float32),
        scratch_types=[
            pltpu.VMEM((_GCH,), jnp.int32),
            pltpu.VMEM((_GCH, C), jnp.float32),
            pltpu.SemaphoreType.DMA,
        ],
    )
    def k(table_hbm, idx_hbm, out_hbm, idx_v, rows_v, sem):
        wid = lax.axis_index("s") * 2 + lax.axis_index("c")
        base = wid * m_per_w

        def chunk(i, carry):
            off = base + i * _GCH
            pltpu.sync_copy(idx_hbm.at[pl.ds(off, _GCH)], idx_v)
            pltpu.async_copy(table_hbm.at[idx_v], rows_v, sem).wait()
            pltpu.sync_copy(rows_v, out_hbm.at[pl.ds(off, _GCH)])
            return carry

        lax.fori_loop(0, nch, chunk, 0)

    return k(table, idx)


_CAP = 512


def _sc_topk(d, nctr, ncand):
    gpb = nctr // 16
    groups = B * gpb
    gpw = groups // _NW
    nsv = ncand // 256
    gshift = {128: 7, 64: 6}[gpb]

    @functools.partial(
        pl.kernel,
        mesh=_SC_MESH(),
        compiler_params=pltpu.CompilerParams(needs_layout_passes=False),
        out_type=jax.ShapeDtypeStruct((B * nctr, K), jnp.int32),
        scratch_types=[
            pltpu.VMEM((16, ncand), jnp.float32),
            pltpu.VMEM((_CAP + 16,), jnp.float32),
            pltpu.VMEM((_CAP + 16,), jnp.int32),
            pltpu.VMEM((16, K), jnp.int32),
        ],
    )
    def k(d_hbm, out_hbm, dbuf, vbuf, ibuf, obuf):
        wid = lax.axis_index("s") * 2 + lax.axis_index("c")
        lane = lax.iota(jnp.int32, 16)
        inf_v = jnp.full((16,), jnp.inf, jnp.float32)
        zero_i = jnp.zeros((16,), jnp.int32)

        def group_body(gi, carry):
            g = wid * gpw + gi
            b = lax.shift_right_logical(g, gshift)
            i0 = jnp.bitwise_and(g, gpb - 1) * 16
            pltpu.sync_copy(d_hbm.at[b, pl.ds(i0, 16), :], dbuf)
            joff = b * ncand

            def center_body(l, carry2):
                def min_body(i, ms):
                    m0, m1 = ms
                    m0 = jnp.minimum(m0, dbuf[l, pl.ds(i * 32, 16)])
                    m1 = jnp.minimum(m1, dbuf[l, pl.ds(i * 32 + 16, 16)])
                    return (m0, m1)

                m0, m1 = lax.fori_loop(0, ncand // 32, min_body, (inf_v, inf_v))
                sm0 = lax.sort(m0)
                sm1 = lax.sort(m1)
                t = jnp.max(jnp.minimum(sm0, lax.rev(sm1, (0,))))

                def scan_body(i, fill):
                    f = fill
                    for u in range(2):
                        jj = i * 2 + u
                        v = dbuf[l, pl.ds(jj * 16, 16)]
                        msk = v <= t
                        fw = jnp.minimum(f, _CAP)
                        plsc.store_compressed(vbuf.at[pl.ds(fw, 16)], v, mask=msk)
                        iv = lane + (jj * 16 + joff)
                        plsc.store_compressed(ibuf.at[pl.ds(fw, 16)], iv, mask=msk)
                        pc = plsc.all_reduce_population_count(msk)
                        f = f + pc[0]
                    return f

                fill = lax.fori_loop(0, ncand // 32, scan_body, jnp.int32(0))
                vbuf[pl.ds(jnp.minimum(fill, _CAP), 16)] = inf_v

                nch = lax.shift_right_logical(jnp.minimum(fill, _CAP) + 15, 4)

                def ch_body(cc, bvbi):
                    bv, bi = bvbi
                    cv = vbuf[pl.ds(cc * 16, 16)]
                    ci = ibuf[pl.ds(cc * 16, 16)]
                    cvs, cis = plsc.sort_key_val(cv, ci)
                    bvr = lax.rev(bv, (0,))
                    bir = lax.rev(bi, (0,))
                    take = cvs <= bvr
                    nv = jnp.where(take, cvs, bvr)
                    ni = jnp.where(take, cis, bir)
                    r = plsc.sort_key_val(nv, ni)
                    return (r[0], r[1])

                bv, bi = lax.fori_loop(0, nch, ch_body, (inf_v, zero_i))
                obuf[l] = bi
                return carry2

            lax.fori_loop(0, 16, center_body, 0)
            pltpu.sync_copy(obuf, out_hbm.at[pl.ds(g * 16, 16)])
            return carry

        lax.fori_loop(0, gpw, group_body, 0)

    return k(d)


def kernel(position_matrix, channel_matrix, n_select_0, n_select_1, n_select_2, W1_0, b1_0, W1_1, b1_1, Wres1, bres1, ln1_g, ln1_b, W2_0, b2_0, W2_1, b2_1, Wres2, bres2, ln2_g, ln2_b):
    pos = position_matrix
    feat = channel_matrix

    pos8 = jnp.pad(pos, ((0, 0), (0, 0), (0, 5)))
    pos8T = jnp.transpose(pos8, (0, 2, 1))
    pos2_8 = pos8[:, :N1]
    pos2_8T = pos8T[:, :, :N1]
    npos2_8 = pos8[:, :N2]
    f_pad = jnp.concatenate(
        [feat, pos, jnp.zeros((B, N0, 5), jnp.float32)], axis=-1)
    w10p = jnp.pad(W1_0, ((0, 5), (0, 0)))
    w1p8 = jnp.pad(W1_0[C:], ((0, 5), (0, 0)))
    w2p8 = jnp.pad(W2_0[C:], ((0, 5), (0, 0)))
    w2f = W2_0[:C]
    r1 = lambda v: v.reshape(1, C)
    feat1 = feat[:, :N1]

    z1 = _call_z(f_pad, w10p)
    d1 = _call_d(pos2_8, pos8T, N1, N0)
    d2 = _call_d(npos2_8, pos2_8T, N2, N1)
    c1, res1, np2, c2 = _call_p(
        pos2_8, feat1, Wres1, r1(bres1), w1p8, r1(b1_0), w2p8, r1(b2_0))

    idx1 = _sc_topk(d1, N1, N0).reshape(-1)
    g1 = _sc_gather(z1.reshape(B * N0, C), idx1).reshape(B, N1, K * C)

    och = _call_mlp(g1, c1, W1_1, r1(b1_1), r1(ln1_g), r1(ln1_b), N1, 512)

    z2, res2 = _call_b2(och, np2, w2f, Wres2, r1(bres2), res1)

    idx2 = _sc_topk(d2, N2, N1).reshape(-1)
    g2 = _sc_gather(z2.reshape(B * N1, C), idx2).reshape(B, N2, K * C)

    out_ch = _call_c(g2, c2, W2_1, r1(b2_1), r1(ln2_g), r1(ln2_b), res2)
    return (pos[:, :N2], out_ch)


# speedup vs baseline: 6.6420x; 1.0456x over previous
import functools
import jax
import jax.numpy as jnp
from jax import lax
from jax.experimental import pallas as pl
from jax.experimental.pallas import tpu as pltpu
from jax.experimental.pallas import tpu_sc as plsc

B = 4
N0 = 4096
N1 = 2048
N2 = 1024
C = 128
K = 16


def _z_body(f_ref, w10_ref, z_ref):
    z_ref[0] = jnp.dot(f_ref[0], w10_ref[...], preferred_element_type=jnp.float32, precision=lax.Precision.HIGHEST)


def _call_z(f_pad, w10p):
    return pl.pallas_call(
        _z_body,
        grid=(B, N0 // 512),
        in_specs=[
            pl.BlockSpec((1, 512, 136), lambda b, j: (b, j, 0)),
            pl.BlockSpec((136, C), lambda b, j: (0, 0)),
        ],
        out_specs=pl.BlockSpec((1, 512, C), lambda b, j: (b, j, 0)),
        out_shape=jax.ShapeDtypeStruct((B, N0, C), jnp.float32),
    )(f_pad, w10p)


def _d_body(ctr_ref, candT_ref, d_ref):
    p = ctr_ref[0]
    nT = candT_ref[0]
    d = jnp.zeros(d_ref.shape[1:], jnp.float32)
    for c in range(3):
        diff = p[:, c:c + 1] - nT[c:c + 1, :]
        d = d + diff * diff
    d_ref[0] = d


def _call_d(ctr8, candT8, nctr, ncand):
    return pl.pallas_call(
        _d_body,
        grid=(B, nctr // 512),
        in_specs=[
            pl.BlockSpec((1, 512, 8), lambda b, j: (b, j, 0)),
            pl.BlockSpec((1, 8, ncand), lambda b, j: (b, 0, 0)),
        ],
        out_specs=pl.BlockSpec((1, 512, ncand), lambda b, j: (b, j, 0)),
        out_shape=jax.ShapeDtypeStruct((B, nctr, ncand), jnp.float32),
    )(ctr8, candT8)


def _p_body(pos8_ref, feat1_ref, wres1_ref, bres1_ref, w1p_ref,
            b10_ref, w2p_ref, b20_ref, c1_ref, res1_ref, np2_ref, c2_ref):
    jb = pl.program_id(1)
    p = pos8_ref[0]
    c1_ref[0] = b10_ref[...] - jnp.dot(p, w1p_ref[...], preferred_element_type=jnp.float32, precision=lax.Precision.HIGHEST)
    res1_ref[0] = jnp.dot(feat1_ref[0], wres1_ref[...], preferred_element_type=jnp.float32, precision=lax.Precision.HIGHEST) + bres1_ref[...]
    np2 = jnp.dot(p, w2p_ref[...], preferred_element_type=jnp.float32, precision=lax.Precision.HIGHEST)
    np2_ref[0] = np2

    @pl.when(jb == 0)
    def _():
        c2_ref[0] = b20_ref[...] - np2


def _call_p(pos2_8, feat1, wres1, bres1, w1p8, b10, w2p8, b20):
    return pl.pallas_call(
        _p_body,
        grid=(B, 2),
        in_specs=[
            pl.BlockSpec((1, 1024, 8), lambda b, j: (b, j, 0)),
            pl.BlockSpec((1, 1024, C), lambda b, j: (b, j, 0)),
            pl.BlockSpec((C, C), lambda b, j: (0, 0)),
            pl.BlockSpec((1, C), lambda b, j: (0, 0)),
            pl.BlockSpec((8, C), lambda b, j: (0, 0)),
            pl.BlockSpec((1, C), lambda b, j: (0, 0)),
            pl.BlockSpec((8, C), lambda b, j: (0, 0)),
            pl.BlockSpec((1, C), lambda b, j: (0, 0)),
        ],
        out_specs=[
            pl.BlockSpec((1, 1024, C), lambda b, j: (b, j, 0)),
            pl.BlockSpec((1, 1024, C), lambda b, j: (b, j, 0)),
            pl.BlockSpec((1, 1024, C), lambda b, j: (b, j, 0)),
            pl.BlockSpec((1, 1024, C), lambda b, j: (b, 0, 0)),
        ],
        out_shape=[
            jax.ShapeDtypeStruct((B, N1, C), jnp.float32),
            jax.ShapeDtypeStruct((B, N1, C), jnp.float32),
            jax.ShapeDtypeStruct((B, N1, C), jnp.float32),
            jax.ShapeDtypeStruct((B, N2, C), jnp.float32),
        ],
    )(pos2_8, feat1, wres1, bres1, w1p8, b10, w2p8, b20)


def _mlp_body(g_ref, c_ref, w_ref, b_ref, lng_ref, lnb_ref, out_ref):
    cb = c_ref[0]
    w = w_ref[...]
    bb = b_ref[...]
    acc = jnp.zeros(out_ref.shape[1:], jnp.float32)
    for k in range(K):
        hk = jnp.maximum(g_ref[0][:, k * C:(k + 1) * C] + cb, 0.0)
        acc = jnp.maximum(acc, jnp.maximum(jnp.dot(hk, w, preferred_element_type=jnp.float32, precision=lax.Precision.HIGHEST) + bb, 0.0))
    mu = jnp.mean(acc, axis=1, keepdims=True)
    xc = acc - mu
    var = jnp.mean(xc * xc, axis=1, keepdims=True)
    ln = xc * lax.rsqrt(var + 1e-5) * lng_ref[...] + lnb_ref[...]
    out_ref[0] = ln * jax.nn.sigmoid(ln)


def _call_mlp(g, c, w, b, lng, lnb, n_rows, blk):
    return pl.pallas_call(
        _mlp_body,
        grid=(B, n_rows // blk),
        in_specs=[
            pl.BlockSpec((1, blk, K * C), lambda b_, i: (b_, i, 0)),
            pl.BlockSpec((1, blk, C), lambda b_, i: (b_, i, 0)),
            pl.BlockSpec((C, C), lambda b_, i: (0, 0)),
            pl.BlockSpec((1, C), lambda b_, i: (0, 0)),
            pl.BlockSpec((1, C), lambda b_, i: (0, 0)),
            pl.BlockSpec((1, C), lambda b_, i: (0, 0)),
        ],
        out_specs=pl.BlockSpec((1, blk, C), lambda b_, i: (b_, i, 0)),
        out_shape=jax.ShapeDtypeStruct((B, n_rows, C), jnp.float32),
    )(g, c, w, b, lng, lnb)


def _b2_body(och_ref, np2_ref, w2f_ref, wres2_ref, bres2_ref, res1_ref,
             z2_ref, res2_ref):
    och = och_ref[0]
    z2_ref[0] = jnp.dot(och, w2f_ref[...], preferred_element_type=jnp.float32, precision=lax.Precision.HIGHEST) + np2_ref[0]
    res2_ref[0] = (jnp.dot(och[:N2], wres2_ref[...], preferred_element_type=jnp.float32, precision=lax.Precision.HIGHEST)
                   + bres2_ref[...] + res1_ref[0])


def _call_b2(och, np2, w2f, wres2, bres2, res1):
    return pl.pallas_call(
        _b2_body,
        grid=(B,),
        in_specs=[
            pl.BlockSpec((1, N1, C), lambda b: (b, 0, 0)),
            pl.BlockSpec((1, N1, C), lambda b: (b, 0, 0)),
            pl.BlockSpec((C, C), lambda b: (0, 0)),
            pl.BlockSpec((C, C), lambda b: (0, 0)),
            pl.BlockSpec((1, C), lambda b: (0, 0)),
            pl.BlockSpec((1, N2, C), lambda b: (b, 0, 0)),
        ],
        out_specs=[
            pl.BlockSpec((1, N1, C), lambda b: (b, 0, 0)),
            pl.BlockSpec((1, N2, C), lambda b: (b, 0, 0)),
        ],
        out_shape=[
            jax.ShapeDtypeStruct((B, N1, C), jnp.float32),
            jax.ShapeDtypeStruct((B, N2, C), jnp.float32),
        ],
    )(och, np2, w2f, wres2, bres2, res1)


def _c_body(g_ref, c_ref, w_ref, b_ref, lng_ref, lnb_ref, res2_ref, out_ref):
    cb = c_ref[0]
    w = w_ref[...]
    bb = b_ref[...]
    acc = jnp.zeros(out_ref.shape[1:], jnp.float32)
    for k in range(K):
        hk = jnp.maximum(g_ref[0][:, k * C:(k + 1) * C] + cb, 0.0)
        acc = jnp.maximum(acc, jnp.maximum(jnp.dot(hk, w, preferred_element_type=jnp.float32, precision=lax.Precision.HIGHEST) + bb, 0.0))
    mu = jnp.mean(acc, axis=1, keepdims=True)
    xc = acc - mu
    var = jnp.mean(xc * xc, axis=1, keepdims=True)
    ln = xc * lax.rsqrt(var + 1e-5) * lng_ref[...] + lnb_ref[...]
    out_ref[0] = ln * jax.nn.sigmoid(ln) + res2_ref[0]


def _call_c(g2, c2, w21, b21, lng, lnb, res2):
    return pl.pallas_call(
        _c_body,
        grid=(B, 2),
        in_specs=[
            pl.BlockSpec((1, 512, K * C), lambda b_, i: (b_, i, 0)),
            pl.BlockSpec((1, 512, C), lambda b_, i: (b_, i, 0)),
            pl.BlockSpec((C, C), lambda b_, i: (0, 0)),
            pl.BlockSpec((1, C), lambda b_, i: (0, 0)),
            pl.BlockSpec((1, C), lambda b_, i: (0, 0)),
            pl.BlockSpec((1, C), lambda b_, i: (0, 0)),
            pl.BlockSpec((1, 512, C), lambda b_, i: (b_, i, 0)),
        ],
        out_specs=pl.BlockSpec((1, 512, C), lambda b_, i: (b_, i, 0)),
        out_shape=jax.ShapeDtypeStruct((B, N2, C), jnp.float32),
    )(g2, c2, w21, b21, lng, lnb, res2)


_SC_MESH = lambda: plsc.VectorSubcoreMesh(core_axis_name="c", subcore_axis_name="s")
_NW = 32
_GCH = 128


def _sc_gather(table, idx):
    M = idx.shape[0]
    m_per_w = M // _NW
    nch = m_per_w // _GCH

    @functools.partial(
        pl.kernel,
        mesh=_SC_MESH(),
        out_type=jax.ShapeDtypeStruct((M, C), jnp.float32),
        scratch_types=[
            pltpu.VMEM((_GCH,), jnp.int32),
            pltpu.VMEM((_GCH, C), jnp.float32),
            pltpu.SemaphoreType.DMA,
        ],
    )
    def k(table_hbm, idx_hbm, out_hbm, idx_v, rows_v, sem):
        wid = lax.axis_index("s") * 2 + lax.axis_index("c")
        base = wid * m_per_w

        def chunk(i, carry):
            off = base + i * _GCH
            pltpu.sync_copy(idx_hbm.at[pl.ds(off, _GCH)], idx_v)
            pltpu.async_copy(table_hbm.at[idx_v], rows_v, sem).wait()
            pltpu.sync_copy(rows_v, out_hbm.at[pl.ds(off, _GCH)])
            return carry

        lax.fori_loop(0, nch, chunk, 0)

    return k(table, idx)


_CAP = 512


def _sc_topk(d, nctr, ncand):
    rpg = 8
    gpb = nctr // rpg
    groups = B * gpb
    gpw = groups // _NW
    gshift = {256: 8, 128: 7}[gpb]

    @functools.partial(
        pl.kernel,
        mesh=_SC_MESH(),
        compiler_params=pltpu.CompilerParams(needs_layout_passes=False),
        out_type=jax.ShapeDtypeStruct((B * nctr, K), jnp.int32),
        scratch_types=[
            pltpu.VMEM((2, rpg, ncand), jnp.float32),
            pltpu.VMEM((_CAP + 16,), jnp.float32),
            pltpu.VMEM((_CAP + 16,), jnp.int32),
            pltpu.VMEM((rpg, K), jnp.int32),
            pltpu.SemaphoreType.DMA,
        ],
    )
    def k(d_hbm, out_hbm, dbuf2, vbuf, ibuf, obuf, sem):
        wid = lax.axis_index("s") * 2 + lax.axis_index("c")
        lane = lax.iota(jnp.int32, 16)
        inf_v = jnp.full((16,), jnp.inf, jnp.float32)
        zero_i = jnp.zeros((16,), jnp.int32)

        def dma(gi, par):
            g = wid * gpw + gi
            b = lax.shift_right_logical(g, gshift)
            i0 = jnp.bitwise_and(g, gpb - 1) * rpg
            return pltpu.make_async_copy(
                d_hbm.at[b, pl.ds(i0, rpg), :], dbuf2.at[par], sem)

        dma(0, 0).start()

        def group_body(gi, carry):
            g = wid * gpw + gi
            b = lax.shift_right_logical(g, gshift)
            par = jnp.bitwise_and(gi, 1)
            dma(gi, par).wait()

            @pl.when(gi + 1 < gpw)
            def _():
                dma(gi + 1, 1 - par).start()

            dbuf = dbuf2.at[par]
            joff = b * ncand

            def center_body(l, carry2):
                def min_body(i, ms):
                    m0, m1 = ms
                    m0 = jnp.minimum(m0, dbuf[l, pl.ds(i * 32, 16)])
                    m1 = jnp.minimum(m1, dbuf[l, pl.ds(i * 32 + 16, 16)])
                    return (m0, m1)

                m0, m1 = lax.fori_loop(0, ncand // 32, min_body, (inf_v, inf_v))
                sm0 = lax.sort(m0)
                sm1 = lax.sort(m1)
                t = jnp.max(jnp.minimum(sm0, lax.rev(sm1, (0,))))

                def scan_body(i, fill):
                    f = fill
                    for u in range(2):
                        jj = i * 2 + u
                        v = dbuf[l, pl.ds(jj * 16, 16)]
                        msk = v <= t
                        fw = jnp.minimum(f, _CAP)
                        plsc.store_compressed(vbuf.at[pl.ds(fw, 16)], v, mask=msk)
                        iv = lane + (jj * 16 + joff)
                        plsc.store_compressed(ibuf.at[pl.ds(fw, 16)], iv, mask=msk)
                        pc = plsc.all_reduce_population_count(msk)
                        f = f + pc[0]
                    return f

                fill = lax.fori_loop(0, ncand // 32, scan_body, jnp.int32(0))
                vbuf[pl.ds(jnp.minimum(fill, _CAP), 16)] = inf_v

                nch = lax.shift_right_logical(jnp.minimum(fill, _CAP) + 15, 4)

                def ch_body(cc, bvbi):
                    bv, bi = bvbi
                    cv = vbuf[pl.ds(cc * 16, 16)]
                    ci = ibuf[pl.ds(cc * 16, 16)]
                    cvs, cis = plsc.sort_key_val(cv, ci)
                    bvr = lax.rev(bv, (0,))
                    bir = lax.rev(bi, (0,))
                    take = cvs <= bvr
                    nv = jnp.where(take, cvs, bvr)
                    ni = jnp.where(take, cis, bir)
                    r = plsc.sort_key_val(nv, ni)
                    return (r[0], r[1])

                bv, bi = lax.fori_loop(0, nch, ch_body, (inf_v, zero_i))
                obuf[l] = bi
                return carry2

            lax.fori_loop(0, rpg, center_body, 0)
            pltpu.sync_copy(obuf, out_hbm.at[pl.ds(g * rpg, rpg)])
            return carry

        lax.fori_loop(0, gpw, group_body, 0)

    return k(d)


def kernel(position_matrix, channel_matrix, n_select_0, n_select_1, n_select_2, W1_0, b1_0, W1_1, b1_1, Wres1, bres1, ln1_g, ln1_b, W2_0, b2_0, W2_1, b2_1, Wres2, bres2, ln2_g, ln2_b):
    pos = position_matrix
    feat = channel_matrix

    pos8 = jnp.pad(pos, ((0, 0), (0, 0), (0, 5)))
    pos8T = jnp.transpose(pos8, (0, 2, 1))
    pos2_8 = pos8[:, :N1]
    pos2_8T = pos8T[:, :, :N1]
    npos2_8 = pos8[:, :N2]
    f_pad = jnp.concatenate(
        [feat, pos, jnp.zeros((B, N0, 5), jnp.float32)], axis=-1)
    w10p = jnp.pad(W1_0, ((0, 5), (0, 0)))
    w1p8 = jnp.pad(W1_0[C:], ((0, 5), (0, 0)))
    w2p8 = jnp.pad(W2_0[C:], ((0, 5), (0, 0)))
    w2f = W2_0[:C]
    r1 = lambda v: v.reshape(1, C)
    feat1 = feat[:, :N1]

    z1 = _call_z(f_pad, w10p)
    d1 = _call_d(pos2_8, pos8T, N1, N0)
    d2 = _call_d(npos2_8, pos2_8T, N2, N1)
    c1, res1, np2, c2 = _call_p(
        pos2_8, feat1, Wres1, r1(bres1), w1p8, r1(b1_0), w2p8, r1(b2_0))

    idx1 = _sc_topk(d1, N1, N0).reshape(-1)
    g1 = _sc_gather(z1.reshape(B * N0, C), idx1).reshape(B, N1, K * C)

    och = _call_mlp(g1, c1, W1_1, r1(b1_1), r1(ln1_g), r1(ln1_b), N1, 512)

    z2, res2 = _call_b2(och, np2, w2f, Wres2, r1(bres2), res1)

    idx2 = _sc_topk(d2, N2, N1).reshape(-1)
    g2 = _sc_gather(z2.reshape(B * N1, C), idx2).reshape(B, N2, K * C)

    out_ch = _call_c(g2, c2, W2_1, r1(b2_1), r1(ln2_g), r1(ln2_b), res2)
    return (pos[:, :N2], out_ch)


# speedup vs baseline: 7.2208x; 1.0871x over previous
import functools
import jax
import jax.numpy as jnp
from jax import lax
from jax.experimental import pallas as pl
from jax.experimental.pallas import tpu as pltpu
from jax.experimental.pallas import tpu_sc as plsc

B = 4
N0 = 4096
N1 = 2048
N2 = 1024
C = 128
K = 16


def _z_body(f_ref, w10_ref, z_ref):
    z_ref[0] = jnp.dot(f_ref[0], w10_ref[...], preferred_element_type=jnp.float32, precision=lax.Precision.HIGHEST)


def _call_z(f_pad, w10p):
    return pl.pallas_call(
        _z_body,
        grid=(B, N0 // 512),
        in_specs=[
            pl.BlockSpec((1, 512, 136), lambda b, j: (b, j, 0)),
            pl.BlockSpec((136, C), lambda b, j: (0, 0)),
        ],
        out_specs=pl.BlockSpec((1, 512, C), lambda b, j: (b, j, 0)),
        out_shape=jax.ShapeDtypeStruct((B, N0, C), jnp.float32),
    )(f_pad, w10p)


def _d_body(ctr_ref, candT_ref, d_ref):
    p = ctr_ref[0]
    nT = candT_ref[0]
    d = jnp.zeros(d_ref.shape[1:], jnp.float32)
    for c in range(3):
        diff = p[:, c:c + 1] - nT[c:c + 1, :]
        d = d + diff * diff
    d_ref[0] = d


def _call_d(ctr8, candT8, nctr, ncand):
    return pl.pallas_call(
        _d_body,
        grid=(B, nctr // 512),
        in_specs=[
            pl.BlockSpec((1, 512, 8), lambda b, j: (b, j, 0)),
            pl.BlockSpec((1, 8, ncand), lambda b, j: (b, 0, 0)),
        ],
        out_specs=pl.BlockSpec((1, 512, ncand), lambda b, j: (b, j, 0)),
        out_shape=jax.ShapeDtypeStruct((B, nctr, ncand), jnp.float32),
    )(ctr8, candT8)


def _p_body(pos8_ref, feat1_ref, wres1_ref, bres1_ref, w1p_ref,
            b10_ref, w2p_ref, b20_ref, c1_ref, res1_ref, np2_ref, c2_ref):
    jb = pl.program_id(1)
    p = pos8_ref[0]
    c1_ref[0] = b10_ref[...] - jnp.dot(p, w1p_ref[...], preferred_element_type=jnp.float32, precision=lax.Precision.HIGHEST)
    res1_ref[0] = jnp.dot(feat1_ref[0], wres1_ref[...], preferred_element_type=jnp.float32, precision=lax.Precision.HIGHEST) + bres1_ref[...]
    np2 = jnp.dot(p, w2p_ref[...], preferred_element_type=jnp.float32, precision=lax.Precision.HIGHEST)
    np2_ref[0] = np2

    @pl.when(jb == 0)
    def _():
        c2_ref[0] = b20_ref[...] - np2


def _call_p(pos2_8, feat1, wres1, bres1, w1p8, b10, w2p8, b20):
    return pl.pallas_call(
        _p_body,
        grid=(B, 2),
        in_specs=[
            pl.BlockSpec((1, 1024, 8), lambda b, j: (b, j, 0)),
            pl.BlockSpec((1, 1024, C), lambda b, j: (b, j, 0)),
            pl.BlockSpec((C, C), lambda b, j: (0, 0)),
            pl.BlockSpec((1, C), lambda b, j: (0, 0)),
            pl.BlockSpec((8, C), lambda b, j: (0, 0)),
            pl.BlockSpec((1, C), lambda b, j: (0, 0)),
            pl.BlockSpec((8, C), lambda b, j: (0, 0)),
            pl.BlockSpec((1, C), lambda b, j: (0, 0)),
        ],
        out_specs=[
            pl.BlockSpec((1, 1024, C), lambda b, j: (b, j, 0)),
            pl.BlockSpec((1, 1024, C), lambda b, j: (b, j, 0)),
            pl.BlockSpec((1, 1024, C), lambda b, j: (b, j, 0)),
            pl.BlockSpec((1, 1024, C), lambda b, j: (b, 0, 0)),
        ],
        out_shape=[
            jax.ShapeDtypeStruct((B, N1, C), jnp.float32),
            jax.ShapeDtypeStruct((B, N1, C), jnp.float32),
            jax.ShapeDtypeStruct((B, N1, C), jnp.float32),
            jax.ShapeDtypeStruct((B, N2, C), jnp.float32),
        ],
    )(pos2_8, feat1, wres1, bres1, w1p8, b10, w2p8, b20)


def _mlp_body(g_ref, c_ref, w_ref, b_ref, lng_ref, lnb_ref, out_ref):
    cb = c_ref[0]
    w = w_ref[...]
    bb = b_ref[...]
    acc = jnp.zeros(out_ref.shape[1:], jnp.float32)
    for k in range(K):
        hk = jnp.maximum(g_ref[0][:, k * C:(k + 1) * C] + cb, 0.0)
        acc = jnp.maximum(acc, jnp.maximum(jnp.dot(hk, w, preferred_element_type=jnp.float32, precision=lax.Precision.HIGHEST) + bb, 0.0))
    mu = jnp.mean(acc, axis=1, keepdims=True)
    xc = acc - mu
    var = jnp.mean(xc * xc, axis=1, keepdims=True)
    ln = xc * lax.rsqrt(var + 1e-5) * lng_ref[...] + lnb_ref[...]
    out_ref[0] = ln * jax.nn.sigmoid(ln)


def _call_mlp(g, c, w, b, lng, lnb, n_rows, blk):
    return pl.pallas_call(
        _mlp_body,
        grid=(B, n_rows // blk),
        in_specs=[
            pl.BlockSpec((1, blk, K * C), lambda b_, i: (b_, i, 0)),
            pl.BlockSpec((1, blk, C), lambda b_, i: (b_, i, 0)),
            pl.BlockSpec((C, C), lambda b_, i: (0, 0)),
            pl.BlockSpec((1, C), lambda b_, i: (0, 0)),
            pl.BlockSpec((1, C), lambda b_, i: (0, 0)),
            pl.BlockSpec((1, C), lambda b_, i: (0, 0)),
        ],
        out_specs=pl.BlockSpec((1, blk, C), lambda b_, i: (b_, i, 0)),
        out_shape=jax.ShapeDtypeStruct((B, n_rows, C), jnp.float32),
    )(g, c, w, b, lng, lnb)


def _b2_body(och_ref, np2_ref, w2f_ref, wres2_ref, bres2_ref, res1_ref,
             z2_ref, res2_ref):
    och = och_ref[0]
    z2_ref[0] = jnp.dot(och, w2f_ref[...], preferred_element_type=jnp.float32, precision=lax.Precision.HIGHEST) + np2_ref[0]
    res2_ref[0] = (jnp.dot(och[:N2], wres2_ref[...], preferred_element_type=jnp.float32, precision=lax.Precision.HIGHEST)
                   + bres2_ref[...] + res1_ref[0])


def _call_b2(och, np2, w2f, wres2, bres2, res1):
    return pl.pallas_call(
        _b2_body,
        grid=(B,),
        in_specs=[
            pl.BlockSpec((1, N1, C), lambda b: (b, 0, 0)),
            pl.BlockSpec((1, N1, C), lambda b: (b, 0, 0)),
            pl.BlockSpec((C, C), lambda b: (0, 0)),
            pl.BlockSpec((C, C), lambda b: (0, 0)),
            pl.BlockSpec((1, C), lambda b: (0, 0)),
            pl.BlockSpec((1, N2, C), lambda b: (b, 0, 0)),
        ],
        out_specs=[
            pl.BlockSpec((1, N1, C), lambda b: (b, 0, 0)),
            pl.BlockSpec((1, N2, C), lambda b: (b, 0, 0)),
        ],
        out_shape=[
            jax.ShapeDtypeStruct((B, N1, C), jnp.float32),
            jax.ShapeDtypeStruct((B, N2, C), jnp.float32),
        ],
    )(och, np2, w2f, wres2, bres2, res1)


def _c_body(g_ref, c_ref, w_ref, b_ref, lng_ref, lnb_ref, res2_ref, out_ref):
    cb = c_ref[0]
    w = w_ref[...]
    bb = b_ref[...]
    acc = jnp.zeros(out_ref.shape[1:], jnp.float32)
    for k in range(K):
        hk = jnp.maximum(g_ref[0][:, k * C:(k + 1) * C] + cb, 0.0)
        acc = jnp.maximum(acc, jnp.maximum(jnp.dot(hk, w, preferred_element_type=jnp.float32, precision=lax.Precision.HIGHEST) + bb, 0.0))
    mu = jnp.mean(acc, axis=1, keepdims=True)
    xc = acc - mu
    var = jnp.mean(xc * xc, axis=1, keepdims=True)
    ln = xc * lax.rsqrt(var + 1e-5) * lng_ref[...] + lnb_ref[...]
    out_ref[0] = ln * jax.nn.sigmoid(ln) + res2_ref[0]


def _call_c(g2, c2, w21, b21, lng, lnb, res2):
    return pl.pallas_call(
        _c_body,
        grid=(B, 2),
        in_specs=[
            pl.BlockSpec((1, 512, K * C), lambda b_, i: (b_, i, 0)),
            pl.BlockSpec((1, 512, C), lambda b_, i: (b_, i, 0)),
            pl.BlockSpec((C, C), lambda b_, i: (0, 0)),
            pl.BlockSpec((1, C), lambda b_, i: (0, 0)),
            pl.BlockSpec((1, C), lambda b_, i: (0, 0)),
            pl.BlockSpec((1, C), lambda b_, i: (0, 0)),
            pl.BlockSpec((1, 512, C), lambda b_, i: (b_, i, 0)),
        ],
        out_specs=pl.BlockSpec((1, 512, C), lambda b_, i: (b_, i, 0)),
        out_shape=jax.ShapeDtypeStruct((B, N2, C), jnp.float32),
    )(g2, c2, w21, b21, lng, lnb, res2)


_SC_MESH = lambda: plsc.VectorSubcoreMesh(core_axis_name="c", subcore_axis_name="s")
_NW = 32
_GCH = 128


def _sc_gather(table, idx):
    M = idx.shape[0]
    m_per_w = M // _NW
    nch = m_per_w // _GCH

    @functools.partial(
        pl.kernel,
        mesh=_SC_MESH(),
        out_type=jax.ShapeDtypeStruct((M, C), jnp.float32),
        scratch_types=[
            pltpu.VMEM((_GCH,), jnp.int32),
            pltpu.VMEM((_GCH, C), jnp.float32),
            pltpu.SemaphoreType.DMA,
        ],
    )
    def k(table_hbm, idx_hbm, out_hbm, idx_v, rows_v, sem):
        wid = lax.axis_index("s") * 2 + lax.axis_index("c")
        base = wid * m_per_w

        def chunk(i, carry):
            off = base + i * _GCH
            pltpu.sync_copy(idx_hbm.at[pl.ds(off, _GCH)], idx_v)
            pltpu.async_copy(table_hbm.at[idx_v], rows_v, sem).wait()
            pltpu.sync_copy(rows_v, out_hbm.at[pl.ds(off, _GCH)])
            return carry

        lax.fori_loop(0, nch, chunk, 0)

    return k(table, idx)


_CAP = 512


def _sc_topk(d, nctr, ncand):
    rpg = 8
    gpb = nctr // rpg
    groups = B * gpb
    gpw = groups // _NW
    gshift = {256: 8, 128: 7}[gpb]

    @functools.partial(
        pl.kernel,
        mesh=_SC_MESH(),
        compiler_params=pltpu.CompilerParams(needs_layout_passes=False),
        out_type=jax.ShapeDtypeStruct((B * nctr, K), jnp.int32),
        scratch_types=[
            pltpu.VMEM((2, rpg, ncand), jnp.float32),
            pltpu.VMEM((_CAP + 16,), jnp.int32),
            pltpu.VMEM((rpg, K), jnp.int32),
            pltpu.SemaphoreType.DMA,
        ],
    )
    def k(d_hbm, out_hbm, dbuf2, ibuf, obuf, sem):
        wid = lax.axis_index("s") * 2 + lax.axis_index("c")
        lane = lax.iota(jnp.int32, 16)
        inf_v = jnp.full((16,), jnp.inf, jnp.float32)
        zero_i = jnp.zeros((16,), jnp.int32)

        def dma(gi, par):
            g = wid * gpw + gi
            b = lax.shift_right_logical(g, gshift)
            i0 = jnp.bitwise_and(g, gpb - 1) * rpg
            return pltpu.make_async_copy(
                d_hbm.at[b, pl.ds(i0, rpg), :], dbuf2.at[par], sem)

        dma(0, 0).start()

        def group_body(gi, carry):
            g = wid * gpw + gi
            b = lax.shift_right_logical(g, gshift)
            par = jnp.bitwise_and(gi, 1)
            dma(gi, par).wait()

            @pl.when(gi + 1 < gpw)
            def _():
                dma(gi + 1, 1 - par).start()

            dbuf = dbuf2.at[par]
            joff = b * ncand

            def center_body(l, carry2):
                def min_body(i, ms):
                    m0, m1 = ms
                    for u in range(2):
                        m0 = jnp.minimum(m0, dbuf[l, pl.ds((i * 2 + u) * 32, 16)])
                        m1 = jnp.minimum(m1, dbuf[l, pl.ds((i * 2 + u) * 32 + 16, 16)])
                    return (m0, m1)

                m0, m1 = lax.fori_loop(0, ncand // 64, min_body, (inf_v, inf_v))
                sm0 = lax.sort(m0)
                sm1 = lax.sort(m1)
                t = jnp.max(jnp.minimum(sm0, lax.rev(sm1, (0,))))

                def scan_body(i, fill):
                    f = fill
                    for u in range(4):
                        jj = i * 4 + u
                        v = dbuf[l, pl.ds(jj * 16, 16)]
                        msk = v <= t
                        fw = jnp.minimum(f, _CAP)
                        iv = lane + (jj * 16 + joff)
                        plsc.store_compressed(ibuf.at[pl.ds(fw, 16)], iv, mask=msk)
                        pc = plsc.all_reduce_population_count(msk)
                        f = f + pc[0]
                    return f

                fill = lax.fori_loop(0, ncand // 64, scan_body, jnp.int32(0))

                nch = lax.shift_right_logical(jnp.minimum(fill, _CAP) + 15, 4)
                par_s = zero_i + par
                l_s = zero_i + l

                def ch_body(cc, bvbi):
                    bv, bi = bvbi
                    ci = ibuf[pl.ds(cc * 16, 16)]
                    jl = jnp.bitwise_and(ci - joff, ncand - 1)
                    valid = (cc * 16 + lane) < fill
                    cv = plsc.load_gather(dbuf2, [par_s, l_s, jl], mask=valid)
                    cv = jnp.where(valid, cv, jnp.inf)
                    cvs, cis = plsc.sort_key_val(cv, ci)
                    bvr = lax.rev(bv, (0,))
                    bir = lax.rev(bi, (0,))
                    take = cvs <= bvr
                    nv = jnp.where(take, cvs, bvr)
                    ni = jnp.where(take, cis, bir)
                    r = plsc.sort_key_val(nv, ni)
                    return (r[0], r[1])

                bv, bi = lax.fori_loop(0, nch, ch_body, (inf_v, zero_i))
                obuf[l] = bi
                return carry2

            lax.fori_loop(0, rpg, center_body, 0)
            pltpu.sync_copy(obuf, out_hbm.at[pl.ds(g * rpg, rpg)])
            return carry

        lax.fori_loop(0, gpw, group_body, 0)

    return k(d)


def kernel(position_matrix, channel_matrix, n_select_0, n_select_1, n_select_2, W1_0, b1_0, W1_1, b1_1, Wres1, bres1, ln1_g, ln1_b, W2_0, b2_0, W2_1, b2_1, Wres2, bres2, ln2_g, ln2_b):
    pos = position_matrix
    feat = channel_matrix

    pos8 = jnp.pad(pos, ((0, 0), (0, 0), (0, 5)))
    pos8T = jnp.transpose(pos8, (0, 2, 1))
    pos2_8 = pos8[:, :N1]
    pos2_8T = pos8T[:, :, :N1]
    npos2_8 = pos8[:, :N2]
    f_pad = jnp.concatenate(
        [feat, pos, jnp.zeros((B, N0, 5), jnp.float32)], axis=-1)
    w10p = jnp.pad(W1_0, ((0, 5), (0, 0)))
    w1p8 = jnp.pad(W1_0[C:], ((0, 5), (0, 0)))
    w2p8 = jnp.pad(W2_0[C:], ((0, 5), (0, 0)))
    w2f = W2_0[:C]
    r1 = lambda v: v.reshape(1, C)
    feat1 = feat[:, :N1]

    z1 = _call_z(f_pad, w10p)
    d1 = _call_d(pos2_8, pos8T, N1, N0)
    d2 = _call_d(npos2_8, pos2_8T, N2, N1)
    c1, res1, np2, c2 = _call_p(
        pos2_8, feat1, Wres1, r1(bres1), w1p8, r1(b1_0), w2p8, r1(b2_0))

    idx1 = _sc_topk(d1, N1, N0).reshape(-1)
    g1 = _sc_gather(z1.reshape(B * N0, C), idx1).reshape(B, N1, K * C)

    och = _call_mlp(g1, c1, W1_1, r1(b1_1), r1(ln1_g), r1(ln1_b), N1, 512)

    z2, res2 = _call_b2(och, np2, w2f, Wres2, r1(bres2), res1)

    idx2 = _sc_topk(d2, N2, N1).reshape(-1)
    g2 = _sc_gather(z2.reshape(B * N1, C), idx2).reshape(B, N2, K * C)

    out_ch = _call_c(g2, c2, W2_1, r1(b2_1), r1(ln2_g), r1(ln2_b), res2)
    return (pos[:, :N2], out_ch)


# speedup vs baseline: 7.7111x; 1.0679x over previous
import functools
import jax
import jax.numpy as jnp
from jax import lax
from jax.experimental import pallas as pl
from jax.experimental.pallas import tpu as pltpu
from jax.experimental.pallas import tpu_sc as plsc

B = 4
N0 = 4096
N1 = 2048
N2 = 1024
C = 128
K = 16


def _z_body(f_ref, w10_ref, z_ref):
    z_ref[0] = jnp.dot(f_ref[0], w10_ref[...], preferred_element_type=jnp.float32, precision=lax.Precision.HIGHEST)


def _call_z(f_pad, w10p):
    return pl.pallas_call(
        _z_body,
        grid=(B, N0 // 512),
        in_specs=[
            pl.BlockSpec((1, 512, 136), lambda b, j: (b, j, 0)),
            pl.BlockSpec((136, C), lambda b, j: (0, 0)),
        ],
        out_specs=pl.BlockSpec((1, 512, C), lambda b, j: (b, j, 0)),
        out_shape=jax.ShapeDtypeStruct((B, N0, C), jnp.float32),
    )(f_pad, w10p)


def _d_body(ctr_ref, candT_ref, d_ref):
    p = ctr_ref[0]
    nT = candT_ref[0]
    d = jnp.zeros(d_ref.shape[1:], jnp.float32)
    for c in range(3):
        diff = p[:, c:c + 1] - nT[c:c + 1, :]
        d = d + diff * diff
    d_ref[0] = d


def _call_d(ctr8, candT8, nctr, ncand):
    return pl.pallas_call(
        _d_body,
        grid=(B, nctr // 512),
        in_specs=[
            pl.BlockSpec((1, 512, 8), lambda b, j: (b, j, 0)),
            pl.BlockSpec((1, 8, ncand), lambda b, j: (b, 0, 0)),
        ],
        out_specs=pl.BlockSpec((1, 512, ncand), lambda b, j: (b, j, 0)),
        out_shape=jax.ShapeDtypeStruct((B, nctr, ncand), jnp.float32),
    )(ctr8, candT8)


def _p_body(pos8_ref, feat1_ref, wres1_ref, bres1_ref, w1p_ref,
            b10_ref, w2p_ref, b20_ref, c1_ref, res1_ref, np2_ref, c2_ref):
    jb = pl.program_id(1)
    p = pos8_ref[0]
    c1_ref[0] = b10_ref[...] - jnp.dot(p, w1p_ref[...], preferred_element_type=jnp.float32, precision=lax.Precision.HIGHEST)
    res1_ref[0] = jnp.dot(feat1_ref[0], wres1_ref[...], preferred_element_type=jnp.float32, precision=lax.Precision.HIGHEST) + bres1_ref[...]
    np2 = jnp.dot(p, w2p_ref[...], preferred_element_type=jnp.float32, precision=lax.Precision.HIGHEST)
    np2_ref[0] = np2

    @pl.when(jb == 0)
    def _():
        c2_ref[0] = b20_ref[...] - np2


def _call_p(pos2_8, feat1, wres1, bres1, w1p8, b10, w2p8, b20):
    return pl.pallas_call(
        _p_body,
        grid=(B, 2),
        in_specs=[
            pl.BlockSpec((1, 1024, 8), lambda b, j: (b, j, 0)),
            pl.BlockSpec((1, 1024, C), lambda b, j: (b, j, 0)),
            pl.BlockSpec((C, C), lambda b, j: (0, 0)),
            pl.BlockSpec((1, C), lambda b, j: (0, 0)),
            pl.BlockSpec((8, C), lambda b, j: (0, 0)),
            pl.BlockSpec((1, C), lambda b, j: (0, 0)),
            pl.BlockSpec((8, C), lambda b, j: (0, 0)),
            pl.BlockSpec((1, C), lambda b, j: (0, 0)),
        ],
        out_specs=[
            pl.BlockSpec((1, 1024, C), lambda b, j: (b, j, 0)),
            pl.BlockSpec((1, 1024, C), lambda b, j: (b, j, 0)),
            pl.BlockSpec((1, 1024, C), lambda b, j: (b, j, 0)),
            pl.BlockSpec((1, 1024, C), lambda b, j: (b, 0, 0)),
        ],
        out_shape=[
            jax.ShapeDtypeStruct((B, N1, C), jnp.float32),
            jax.ShapeDtypeStruct((B, N1, C), jnp.float32),
            jax.ShapeDtypeStruct((B, N1, C), jnp.float32),
            jax.ShapeDtypeStruct((B, N2, C), jnp.float32),
        ],
    )(pos2_8, feat1, wres1, bres1, w1p8, b10, w2p8, b20)


def _mlp_body(g_ref, c_ref, w_ref, b_ref, lng_ref, lnb_ref, out_ref):
    cb = c_ref[0]
    w = w_ref[...]
    bb = b_ref[...]
    acc = jnp.zeros(out_ref.shape[1:], jnp.float32)
    for k in range(K):
        hk = jnp.maximum(g_ref[0][:, k * C:(k + 1) * C] + cb, 0.0)
        acc = jnp.maximum(acc, jnp.maximum(jnp.dot(hk, w, preferred_element_type=jnp.float32, precision=lax.Precision.HIGHEST) + bb, 0.0))
    mu = jnp.mean(acc, axis=1, keepdims=True)
    xc = acc - mu
    var = jnp.mean(xc * xc, axis=1, keepdims=True)
    ln = xc * lax.rsqrt(var + 1e-5) * lng_ref[...] + lnb_ref[...]
    out_ref[0] = ln * jax.nn.sigmoid(ln)


def _call_mlp(g, c, w, b, lng, lnb, n_rows, blk):
    return pl.pallas_call(
        _mlp_body,
        grid=(B, n_rows // blk),
        in_specs=[
            pl.BlockSpec((1, blk, K * C), lambda b_, i: (b_, i, 0)),
            pl.BlockSpec((1, blk, C), lambda b_, i: (b_, i, 0)),
            pl.BlockSpec((C, C), lambda b_, i: (0, 0)),
            pl.BlockSpec((1, C), lambda b_, i: (0, 0)),
            pl.BlockSpec((1, C), lambda b_, i: (0, 0)),
            pl.BlockSpec((1, C), lambda b_, i: (0, 0)),
        ],
        out_specs=pl.BlockSpec((1, blk, C), lambda b_, i: (b_, i, 0)),
        out_shape=jax.ShapeDtypeStruct((B, n_rows, C), jnp.float32),
    )(g, c, w, b, lng, lnb)


def _b2_body(och_ref, np2_ref, w2f_ref, wres2_ref, bres2_ref, res1_ref,
             z2_ref, res2_ref):
    och = och_ref[0]
    z2_ref[0] = jnp.dot(och, w2f_ref[...], preferred_element_type=jnp.float32, precision=lax.Precision.HIGHEST) + np2_ref[0]
    res2_ref[0] = (jnp.dot(och[:N2], wres2_ref[...], preferred_element_type=jnp.float32, precision=lax.Precision.HIGHEST)
                   + bres2_ref[...] + res1_ref[0])


def _call_b2(och, np2, w2f, wres2, bres2, res1):
    return pl.pallas_call(
        _b2_body,
        grid=(B,),
        in_specs=[
            pl.BlockSpec((1, N1, C), lambda b: (b, 0, 0)),
            pl.BlockSpec((1, N1, C), lambda b: (b, 0, 0)),
            pl.BlockSpec((C, C), lambda b: (0, 0)),
            pl.BlockSpec((C, C), lambda b: (0, 0)),
            pl.BlockSpec((1, C), lambda b: (0, 0)),
            pl.BlockSpec((1, N2, C), lambda b: (b, 0, 0)),
        ],
        out_specs=[
            pl.BlockSpec((1, N1, C), lambda b: (b, 0, 0)),
            pl.BlockSpec((1, N2, C), lambda b: (b, 0, 0)),
        ],
        out_shape=[
            jax.ShapeDtypeStruct((B, N1, C), jnp.float32),
            jax.ShapeDtypeStruct((B, N2, C), jnp.float32),
        ],
    )(och, np2, w2f, wres2, bres2, res1)


def _c_body(g_ref, c_ref, w_ref, b_ref, lng_ref, lnb_ref, res2_ref, out_ref):
    cb = c_ref[0]
    w = w_ref[...]
    bb = b_ref[...]
    acc = jnp.zeros(out_ref.shape[1:], jnp.float32)
    for k in range(K):
        hk = jnp.maximum(g_ref[0][:, k * C:(k + 1) * C] + cb, 0.0)
        acc = jnp.maximum(acc, jnp.maximum(jnp.dot(hk, w, preferred_element_type=jnp.float32, precision=lax.Precision.HIGHEST) + bb, 0.0))
    mu = jnp.mean(acc, axis=1, keepdims=True)
    xc = acc - mu
    var = jnp.mean(xc * xc, axis=1, keepdims=True)
    ln = xc * lax.rsqrt(var + 1e-5) * lng_ref[...] + lnb_ref[...]
    out_ref[0] = ln * jax.nn.sigmoid(ln) + res2_ref[0]


def _call_c(g2, c2, w21, b21, lng, lnb, res2):
    return pl.pallas_call(
        _c_body,
        grid=(B, 2),
        in_specs=[
            pl.BlockSpec((1, 512, K * C), lambda b_, i: (b_, i, 0)),
            pl.BlockSpec((1, 512, C), lambda b_, i: (b_, i, 0)),
            pl.BlockSpec((C, C), lambda b_, i: (0, 0)),
            pl.BlockSpec((1, C), lambda b_, i: (0, 0)),
            pl.BlockSpec((1, C), lambda b_, i: (0, 0)),
            pl.BlockSpec((1, C), lambda b_, i: (0, 0)),
            pl.BlockSpec((1, 512, C), lambda b_, i: (b_, i, 0)),
        ],
        out_specs=pl.BlockSpec((1, 512, C), lambda b_, i: (b_, i, 0)),
        out_shape=jax.ShapeDtypeStruct((B, N2, C), jnp.float32),
    )(g2, c2, w21, b21, lng, lnb, res2)


_SC_MESH = lambda: plsc.VectorSubcoreMesh(core_axis_name="c", subcore_axis_name="s")
_NW = 32
_GCH = 128


def _sc_gather(table, idx):
    M = idx.shape[0]
    m_per_w = M // _NW
    nch = m_per_w // _GCH

    @functools.partial(
        pl.kernel,
        mesh=_SC_MESH(),
        out_type=jax.ShapeDtypeStruct((M, C), jnp.float32),
        scratch_types=[
            pltpu.VMEM((_GCH,), jnp.int32),
            pltpu.VMEM((_GCH, C), jnp.float32),
            pltpu.SemaphoreType.DMA,
        ],
    )
    def k(table_hbm, idx_hbm, out_hbm, idx_v, rows_v, sem):
        wid = lax.axis_index("s") * 2 + lax.axis_index("c")
        base = wid * m_per_w

        def chunk(i, carry):
            off = base + i * _GCH
            pltpu.sync_copy(idx_hbm.at[pl.ds(off, _GCH)], idx_v)
            pltpu.async_copy(table_hbm.at[idx_v], rows_v, sem).wait()
            pltpu.sync_copy(rows_v, out_hbm.at[pl.ds(off, _GCH)])
            return carry

        lax.fori_loop(0, nch, chunk, 0)

    return k(table, idx)


_CAP = 512


def _sc_topk(d, nctr, ncand):
    rpg = 8
    gpb = nctr // rpg
    groups = B * gpb
    gpw = groups // _NW
    gshift = {256: 8, 128: 7}[gpb]

    @functools.partial(
        pl.kernel,
        mesh=_SC_MESH(),
        compiler_params=pltpu.CompilerParams(needs_layout_passes=False),
        out_type=jax.ShapeDtypeStruct((B * nctr, K), jnp.int32),
        scratch_types=[
            pltpu.VMEM((2, rpg, ncand), jnp.float32),
            pltpu.VMEM((ncand + 16,), jnp.int32),
            pltpu.VMEM((rpg, K), jnp.int32),
            pltpu.SemaphoreType.DMA,
        ],
    )
    def k(d_hbm, out_hbm, dbuf2, ibuf, obuf, sem):
        wid = lax.axis_index("s") * 2 + lax.axis_index("c")
        lane = lax.iota(jnp.int32, 16)
        inf_v = jnp.full((16,), jnp.inf, jnp.float32)
        zero_i = jnp.zeros((16,), jnp.int32)

        def dma(gi, par):
            g = wid * gpw + gi
            b = lax.shift_right_logical(g, gshift)
            i0 = jnp.bitwise_and(g, gpb - 1) * rpg
            return pltpu.make_async_copy(
                d_hbm.at[b, pl.ds(i0, rpg), :], dbuf2.at[par], sem)

        dma(0, 0).start()

        def group_body(gi, carry):
            g = wid * gpw + gi
            b = lax.shift_right_logical(g, gshift)
            par = jnp.bitwise_and(gi, 1)
            dma(gi, par).wait()

            @pl.when(gi + 1 < gpw)
            def _():
                dma(gi + 1, 1 - par).start()

            dbuf = dbuf2.at[par]
            joff = b * ncand

            def center_body(l, carry2):
                def min_body(i, ms):
                    m0, m1 = ms
                    for u in range(2):
                        m0 = jnp.minimum(m0, dbuf[l, pl.ds((i * 2 + u) * 32, 16)])
                        m1 = jnp.minimum(m1, dbuf[l, pl.ds((i * 2 + u) * 32 + 16, 16)])
                    return (m0, m1)

                m0, m1 = lax.fori_loop(0, ncand // 64, min_body, (inf_v, inf_v))
                sm0 = lax.sort(m0)
                sm1 = lax.sort(m1)
                t = jnp.max(jnp.minimum(sm0, lax.rev(sm1, (0,))))

                def scan_body(i, fill):
                    f = fill
                    for u in range(8):
                        jj = i * 8 + u
                        v = dbuf[l, pl.ds(jj * 16, 16)]
                        msk = v <= t
                        iv = lane + (jj * 16 + joff)
                        plsc.store_compressed(ibuf.at[pl.ds(f, 16)], iv, mask=msk)
                        pc = plsc.all_reduce_population_count(msk)
                        f = f + pc[0]
                    return f

                fill = lax.fori_loop(0, ncand // 128, scan_body, jnp.int32(0))

                nch = lax.shift_right_logical(jnp.minimum(fill, _CAP) + 15, 4)
                par_s = zero_i + par
                l_s = zero_i + l

                def ch_body(cc, bvbi):
                    bv, bi = bvbi
                    ci = ibuf[pl.ds(cc * 16, 16)]
                    jl = jnp.bitwise_and(ci - joff, ncand - 1)
                    valid = (cc * 16 + lane) < fill
                    cv = plsc.load_gather(dbuf2, [par_s, l_s, jl], mask=valid)
                    cv = jnp.where(valid, cv, jnp.inf)
                    cvs, cis = plsc.sort_key_val(cv, ci)
                    bvr = lax.rev(bv, (0,))
                    bir = lax.rev(bi, (0,))
                    take = cvs <= bvr
                    nv = jnp.where(take, cvs, bvr)
                    ni = jnp.where(take, cis, bir)
                    r = plsc.sort_key_val(nv, ni)
                    return (r[0], r[1])

                bv, bi = lax.fori_loop(0, nch, ch_body, (inf_v, zero_i))
                obuf[l] = bi
                return carry2

            lax.fori_loop(0, rpg, center_body, 0)
            pltpu.sync_copy(obuf, out_hbm.at[pl.ds(g * rpg, rpg)])
            return carry

        lax.fori_loop(0, gpw, group_body, 0)

    return k(d)


def kernel(position_matrix, channel_matrix, n_select_0, n_select_1, n_select_2, W1_0, b1_0, W1_1, b1_1, Wres1, bres1, ln1_g, ln1_b, W2_0, b2_0, W2_1, b2_1, Wres2, bres2, ln2_g, ln2_b):
    pos = position_matrix
    feat = channel_matrix

    pos8 = jnp.pad(pos, ((0, 0), (0, 0), (0, 5)))
    pos8T = jnp.transpose(pos8, (0, 2, 1))
    pos2_8 = pos8[:, :N1]
    pos2_8T = pos8T[:, :, :N1]
    npos2_8 = pos8[:, :N2]
    f_pad = jnp.concatenate(
        [feat, pos, jnp.zeros((B, N0, 5), jnp.float32)], axis=-1)
    w10p = jnp.pad(W1_0, ((0, 5), (0, 0)))
    w1p8 = jnp.pad(W1_0[C:], ((0, 5), (0, 0)))
    w2p8 = jnp.pad(W2_0[C:], ((0, 5), (0, 0)))
    w2f = W2_0[:C]
    r1 = lambda v: v.reshape(1, C)
    feat1 = feat[:, :N1]

    z1 = _call_z(f_pad, w10p)
    d1 = _call_d(pos2_8, pos8T, N1, N0)
    d2 = _call_d(npos2_8, pos2_8T, N2, N1)
    c1, res1, np2, c2 = _call_p(
        pos2_8, feat1, Wres1, r1(bres1), w1p8, r1(b1_0), w2p8, r1(b2_0))

    idx1 = _sc_topk(d1, N1, N0).reshape(-1)
    g1 = _sc_gather(z1.reshape(B * N0, C), idx1).reshape(B, N1, K * C)

    och = _call_mlp(g1, c1, W1_1, r1(b1_1), r1(ln1_g), r1(ln1_b), N1, 512)

    z2, res2 = _call_b2(och, np2, w2f, Wres2, r1(bres2), res1)

    idx2 = _sc_topk(d2, N2, N1).reshape(-1)
    g2 = _sc_gather(z2.reshape(B * N1, C), idx2).reshape(B, N2, K * C)

    out_ch = _call_c(g2, c2, W2_1, r1(b2_1), r1(ln2_g), r1(ln2_b), res2)
    return (pos[:, :N2], out_ch)
